# Initial kernel scaffold; baseline (speedup 1.0000x reference)
import jax
import jax.numpy as jnp
from jax.experimental import pallas as pl


def kernel(x, edge_index, edge_attr, lin_in_W, lin_in_b, g1_Wl, g1_bl, g1_Wr, g1_br, g1_We, g1_att, g1_bias, res1_W, res1_b, bn1_g, bn1_b, g2_Wl, g2_bl, g2_Wr, g2_br, g2_We, g2_att, g2_bias, res2_W, res2_b, bn2_g, bn2_b, out_W, out_b):
    raise NotImplementedError("write your pallas kernel here")


# speedup vs baseline: 1.0149x; 1.0149x over previous
import jax
import jax.numpy as jnp
from jax.experimental import pallas as pl
from jax.experimental.pallas import tpu as pltpu

N, E, F_IN, ED, H, HC = 10000, 320000, 128, 16, 4, 64


def _lin_in_body(x_ref, w_ref, b_ref, o_ref):
    o_ref[...] = jax.nn.relu(
        jnp.dot(x_ref[...], w_ref[...], preferred_element_type=jnp.float32)
        + b_ref[...]
    )


def _lin_in(x, w, b):
    return pl.pallas_call(
        _lin_in_body,
        out_shape=jax.ShapeDtypeStruct((N, HC), jnp.float32),
        grid=(25,),
        in_specs=[
            pl.BlockSpec((400, F_IN), lambda i: (i, 0)),
            pl.BlockSpec((F_IN, HC), lambda i: (0, 0)),
            pl.BlockSpec((HC,), lambda i: (0,)),
        ],
        out_specs=pl.BlockSpec((400, HC), lambda i: (i, 0)),
    )(x, w, b)


def _gatv2(x, src, dst, ea, Wl, bl, Wr, br, We, att, bias, heads, out_ch, concat):
    n = x.shape[0]
    e = src.shape[0]
    deg = jax.ops.segment_sum(jnp.ones((e,), jnp.float32), dst, n)
    loop_attr = jax.ops.segment_sum(ea, dst, n) / jnp.clip(deg, 1.0)[:, None]
    loop = jnp.arange(n, dtype=src.dtype)
    src2 = jnp.concatenate([src, loop])
    dst2 = jnp.concatenate([dst, loop])
    ea2 = jnp.concatenate([ea, loop_attr], axis=0)
    xl = (x @ Wl + bl).reshape(n, heads, out_ch)
    xr = (x @ Wr + br).reshape(n, heads, out_ch)
    ee = (ea2 @ We).reshape(-1, heads, out_ch)
    xj = xl[src2]
    xi = xr[dst2]
    h = jax.nn.leaky_relu(xj + xi + ee, 0.2)
    alpha = jnp.sum(h * att[None], axis=-1)
    m = jax.ops.segment_max(alpha, dst2, n)
    a = jnp.exp(alpha - m[dst2])
    den = jax.ops.segment_sum(a, dst2, n)
    a = a / (den[dst2] + 1e-16)
    out = jax.ops.segment_sum(xj * a[..., None], dst2, n)
    out = out.reshape(n, heads * out_ch) if concat else out.mean(axis=1)
    return out + bias


def _bn(x, g, b):
    return g * x / jnp.sqrt(1.0 + 1e-5) + b


def kernel(x, edge_index, edge_attr, lin_in_W, lin_in_b, g1_Wl, g1_bl, g1_Wr, g1_br, g1_We, g1_att, g1_bias, res1_W, res1_b, bn1_g, bn1_b, g2_Wl, g2_bl, g2_Wr, g2_br, g2_We, g2_att, g2_bias, res2_W, res2_b, bn2_g, bn2_b, out_W, out_b):
    src, dst = edge_index[0], edge_index[1]
    h = _lin_in(x, lin_in_W, lin_in_b)
    x1 = _gatv2(h, src, dst, edge_attr, g1_Wl, g1_bl, g1_Wr, g1_br, g1_We, g1_att, g1_bias, H, HC, True)
    h = jax.nn.relu(_bn(x1 + h @ res1_W + res1_b, bn1_g, bn1_b))
    x2 = _gatv2(h, src, dst, edge_attr, g2_Wl, g2_bl, g2_Wr, g2_br, g2_We, g2_att, g2_bias, 1, HC, False)
    h = jax.nn.relu(_bn(x2 + h @ res2_W + res2_b, bn2_g, bn2_b))
    return h @ out_W + out_b


# speedup vs baseline: 13.0834x; 12.8913x over previous
import functools

import jax
import jax.numpy as jnp
from jax import lax
from jax.experimental import pallas as pl
from jax.experimental.pallas import tpu as pltpu
from jax.experimental.pallas import tpu_sc as plsc

N, E, F_IN, ED, H, HC = 10000, 320000, 128, 16, 4, 64
F1 = H * HC
W1 = 144
W2 = 80
NC, NS = 2, 16
NP = 10240
NPW = NP // NS
EPW = E // (NC * NS)
EPW1 = E // NS
CH_DEG, CH_G1, CH_G2, CH_S1, CH_S2 = 1000, 200, 400, 200, 400
RSQ = 1.0 / (1.0 + 1e-5) ** 0.5


def _mesh():
    return plsc.VectorSubcoreMesh(
        core_axis_name="c", subcore_axis_name="s",
        num_cores=NC, num_subcores=NS)


_SC_PARAMS = pltpu.CompilerParams(use_tc_tiling_on_sc=False)


def _k_in_body(x_ref, w_ref, b_ref, o_ref):
    o_ref[...] = jax.nn.relu(
        jnp.dot(x_ref[...], w_ref[...], preferred_element_type=jnp.float32)
        + b_ref[...])


def _k_in(x, w, b):
    return pl.pallas_call(
        _k_in_body,
        out_shape=jax.ShapeDtypeStruct((N, HC), jnp.float32),
        grid=(25,),
        in_specs=[pl.BlockSpec((400, F_IN), lambda i: (i, 0)),
                  pl.BlockSpec((F_IN, HC), lambda i: (0, 0)),
                  pl.BlockSpec((1, HC), lambda i: (0, 0))],
        out_specs=pl.BlockSpec((400, HC), lambda i: (i, 0)),
    )(x, w, b)


def _k_lin_body(h_ref, wl_ref, bl_ref, wr_ref, br_ref, xl_ref, xr_ref):
    h = h_ref[...]
    xl_ref[...] = jnp.dot(h, wl_ref[...], preferred_element_type=jnp.float32) + bl_ref[...]
    xr_ref[...] = jnp.dot(h, wr_ref[...], preferred_element_type=jnp.float32) + br_ref[...]


def _k_lin(h, wl, bl, wr, br):
    fin, fout = wl.shape
    return pl.pallas_call(
        _k_lin_body,
        out_shape=(jax.ShapeDtypeStruct((N, fout), jnp.float32),
                   jax.ShapeDtypeStruct((N, fout), jnp.float32)),
        grid=(25,),
        in_specs=[pl.BlockSpec((400, fin), lambda i: (i, 0)),
                  pl.BlockSpec((fin, fout), lambda i: (0, 0)),
                  pl.BlockSpec((1, fout), lambda i: (0, 0)),
                  pl.BlockSpec((fin, fout), lambda i: (0, 0)),
                  pl.BlockSpec((1, fout), lambda i: (0, 0))],
        out_specs=(pl.BlockSpec((400, fout), lambda i: (i, 0)),
                   pl.BlockSpec((400, fout), lambda i: (i, 0))),
    )(h, wl, bl, wr, br)


def _k_edge1_body(xj_ref, xi_ref, ea_ref, we_ref, attv_ref, s4_ref, s4t_ref, o_ref):
    xj = xj_ref[...]
    ee = jnp.dot(ea_ref[...], we_ref[...], preferred_element_type=jnp.float32)
    t = xj + xi_ref[...] + ee
    t = jnp.where(t >= 0, t, 0.2 * t)
    al = jnp.dot(t * attv_ref[...], s4_ref[...], preferred_element_type=jnp.float32)
    s = jnp.exp(al)
    pw = jnp.dot(s, s4t_ref[...], preferred_element_type=jnp.float32) * xj
    eb = pw.shape[0]
    pad = jnp.zeros((eb, W1 - 130), jnp.float32)
    o_ref[0] = jnp.concatenate([pw[:, 0:128], s[:, 0:2], pad], axis=1)
    o_ref[1] = jnp.concatenate([pw[:, 128:256], s[:, 2:4], pad], axis=1)


def _k_edge1(xj, xi, ea, we, attv, s4, s4t):
    eb = 3200
    return pl.pallas_call(
        _k_edge1_body,
        out_shape=jax.ShapeDtypeStruct((NC, E, W1), jnp.float32),
        grid=(E // eb,),
        in_specs=[pl.BlockSpec((eb, F1), lambda i: (i, 0)),
                  pl.BlockSpec((eb, F1), lambda i: (i, 0)),
                  pl.BlockSpec((eb, ED), lambda i: (i, 0)),
                  pl.BlockSpec((ED, F1), lambda i: (0, 0)),
                  pl.BlockSpec((1, F1), lambda i: (0, 0)),
                  pl.BlockSpec((F1, 8), lambda i: (0, 0)),
                  pl.BlockSpec((8, F1), lambda i: (0, 0))],
        out_specs=pl.BlockSpec((NC, eb, W1), lambda i: (0, i, 0)),
    )(xj, xi, ea, we, attv, s4, s4t)


def _k_edge2_body(xj_ref, xi_ref, ea_ref, we_ref, attv_ref, o_ref):
    xj = xj_ref[...]
    ee = jnp.dot(ea_ref[...], we_ref[...], preferred_element_type=jnp.float32)
    t = xj + xi_ref[...] + ee
    t = jnp.where(t >= 0, t, 0.2 * t)
    al = jnp.sum(t * attv_ref[...], axis=1, keepdims=True)
    s = jnp.exp(al)
    pw = s * xj
    eb = pw.shape[0]
    pad = jnp.zeros((eb, W2 - HC - 1), jnp.float32)
    o_ref[...] = jnp.concatenate([pw, s, pad], axis=1)


def _k_edge2(xj, xi, ea, we, attv):
    eb = 8000
    return pl.pallas_call(
        _k_edge2_body,
        out_shape=jax.ShapeDtypeStruct((E, W2), jnp.float32),
        grid=(E // eb,),
        in_specs=[pl.BlockSpec((eb, HC), lambda i: (i, 0)),
                  pl.BlockSpec((eb, HC), lambda i: (i, 0)),
                  pl.BlockSpec((eb, ED), lambda i: (i, 0)),
                  pl.BlockSpec((ED, HC), lambda i: (0, 0)),
                  pl.BlockSpec((1, HC), lambda i: (0, 0))],
        out_specs=pl.BlockSpec((eb, W2), lambda i: (i, 0)),
    )(xj, xi, ea, we, attv)


def _loop_attr_blk(aa0, aa1, ad0, ad1):
    deg = ad0[:, 0:1] + ad1[:, 0:1]
    return (aa0 + aa1) / jnp.maximum(deg, 1.0)


def _k_post1_body(h0_ref, xl_ref, xr_ref, a0_ref, a1_ref, aa0_ref, aa1_ref,
                  ad0_ref, ad1_ref, we_ref, attv_ref, s4_ref, s4t_ref,
                  bias_ref, rw_ref, rb_ref, bg_ref, bb_ref, o_ref):
    xl = xl_ref[...]
    la = _loop_attr_blk(aa0_ref[...], aa1_ref[...], ad0_ref[...], ad1_ref[...])
    eel = jnp.dot(la, we_ref[...], preferred_element_type=jnp.float32)
    t = xl + xr_ref[...] + eel
    t = jnp.where(t >= 0, t, 0.2 * t)
    al = jnp.dot(t * attv_ref[...], s4_ref[...], preferred_element_type=jnp.float32)
    sl = jnp.exp(al)
    a0 = a0_ref[...]
    a1 = a1_ref[...]
    num = jnp.concatenate([a0[:, 0:128], a1[:, 0:128]], axis=1)
    num = num + jnp.dot(sl, s4t_ref[...], preferred_element_type=jnp.float32) * xl
    nb = xl.shape[0]
    den = jnp.concatenate([a0[:, 128:130], a1[:, 128:130],
                           jnp.zeros((nb, 4), jnp.float32)], axis=1) + sl
    denr = jnp.dot(den, s4t_ref[...], preferred_element_type=jnp.float32)
    gat = num / denr + bias_ref[...]
    res = jnp.dot(h0_ref[...], rw_ref[...], preferred_element_type=jnp.float32) + rb_ref[...]
    o_ref[...] = jax.nn.relu(bg_ref[...] * (gat + res) * RSQ + bb_ref[...])


def _k_post1(h0, xl1, xr1, a0, a1, aa0, aa1, ad0, ad1, we, attv, s4, s4t,
             bias, rw, rb, bg, bb):
    nb = 400
    return pl.pallas_call(
        _k_post1_body,
        out_shape=jax.ShapeDtypeStruct((N, F1), jnp.float32),
        grid=(N // nb,),
        in_specs=[pl.BlockSpec((nb, HC), lambda i: (i, 0)),
                  pl.BlockSpec((nb, F1), lambda i: (i, 0)),
                  pl.BlockSpec((nb, F1), lambda i: (i, 0)),
                  pl.BlockSpec((nb, W1), lambda i: (i, 0)),
                  pl.BlockSpec((nb, W1), lambda i: (i, 0)),
                  pl.BlockSpec((nb, ED), lambda i: (i, 0)),
                  pl.BlockSpec((nb, ED), lambda i: (i, 0)),
                  pl.BlockSpec((nb, 8), lambda i: (i, 0)),
                  pl.BlockSpec((nb, 8), lambda i: (i, 0)),
                  pl.BlockSpec((ED, F1), lambda i: (0, 0)),
                  pl.BlockSpec((1, F1), lambda i: (0, 0)),
                  pl.BlockSpec((F1, 8), lambda i: (0, 0)),
                  pl.BlockSpec((8, F1), lambda i: (0, 0)),
                  pl.BlockSpec((1, F1), lambda i: (0, 0)),
                  pl.BlockSpec((HC, F1), lambda i: (0, 0)),
                  pl.BlockSpec((1, F1), lambda i: (0, 0)),
                  pl.BlockSpec((1, F1), lambda i: (0, 0)),
                  pl.BlockSpec((1, F1), lambda i: (0, 0))],
        out_specs=pl.BlockSpec((nb, F1), lambda i: (i, 0)),
    )(h0, xl1, xr1, a0, a1, aa0, aa1, ad0, ad1,
      we, attv, s4, s4t, bias, rw, rb, bg, bb)


def _k_post2_body(h1_ref, xl_ref, xr_ref, a0_ref, a1_ref, aa0_ref, aa1_ref,
                  ad0_ref, ad1_ref, we_ref, attv_ref, bias_ref, rw_ref, rb_ref,
                  bg_ref, bb_ref, ow_ref, ob_ref, o_ref):
    xl = xl_ref[...]
    la = _loop_attr_blk(aa0_ref[...], aa1_ref[...], ad0_ref[...], ad1_ref[...])
    eel = jnp.dot(la, we_ref[...], preferred_element_type=jnp.float32)
    t = xl + xr_ref[...] + eel
    t = jnp.where(t >= 0, t, 0.2 * t)
    al = jnp.sum(t * attv_ref[...], axis=1, keepdims=True)
    sl = jnp.exp(al)
    a0 = a0_ref[...]
    a1 = a1_ref[...]
    num = a0[:, 0:HC] + a1[:, 0:HC] + sl * xl
    den = a0[:, HC:HC + 1] + a1[:, HC:HC + 1] + sl
    gat = num / den + bias_ref[...]
    res = jnp.dot(h1_ref[...], rw_ref[...], preferred_element_type=jnp.float32) + rb_ref[...]
    h2 = jax.nn.relu(bg_ref[...] * (gat + res) * RSQ + bb_ref[...])
    o_ref[...] = jnp.sum(h2 * ow_ref[...], axis=1, keepdims=True) + ob_ref[...]


def _k_post2(h1, xl2, xr2, a0, a1, aa0, aa1, ad0, ad1, we, attv, bias,
             rw, rb, bg, bb, ow, ob):
    nb = 400
    return pl.pallas_call(
        _k_post2_body,
        out_shape=jax.ShapeDtypeStruct((N, 1), jnp.float32),
        grid=(N // nb,),
        in_specs=[pl.BlockSpec((nb, F1), lambda i: (i, 0)),
                  pl.BlockSpec((nb, HC), lambda i: (i, 0)),
                  pl.BlockSpec((nb, HC), lambda i: (i, 0)),
                  pl.BlockSpec((nb, W2), lambda i: (i, 0)),
                  pl.BlockSpec((nb, W2), lambda i: (i, 0)),
                  pl.BlockSpec((nb, ED), lambda i: (i, 0)),
                  pl.BlockSpec((nb, ED), lambda i: (i, 0)),
                  pl.BlockSpec((nb, 8), lambda i: (i, 0)),
                  pl.BlockSpec((nb, 8), lambda i: (i, 0)),
                  pl.BlockSpec((ED, HC), lambda i: (0, 0)),
                  pl.BlockSpec((1, HC), lambda i: (0, 0)),
                  pl.BlockSpec((1, HC), lambda i: (0, 0)),
                  pl.BlockSpec((F1, HC), lambda i: (0, 0)),
                  pl.BlockSpec((1, HC), lambda i: (0, 0)),
                  pl.BlockSpec((1, HC), lambda i: (0, 0)),
                  pl.BlockSpec((1, HC), lambda i: (0, 0)),
                  pl.BlockSpec((1, HC), lambda i: (0, 0)),
                  pl.BlockSpec((1, 1), lambda i: (0, 0))],
        out_specs=pl.BlockSpec((nb, 1), lambda i: (i, 0)),
    )(h1, xl2, xr2, a0, a1, aa0, aa1, ad0, ad1,
      we, attv, bias, rw, rb, bg, bb, ow, ob)


def _deg_body(dst_hbm, ea_hbm, ones_hbm, z16_hbm, z8_hbm, outa_hbm, outd_hbm,
              dst_v, ea_v, ones_v, acc_a, acc_d):
    c = lax.axis_index("c")
    s = lax.axis_index("s")
    wid = c * NS + s
    pltpu.sync_copy(z16_hbm, acc_a.at[pl.ds(s * NPW, NPW)])
    pltpu.sync_copy(z8_hbm, acc_d.at[pl.ds(s * NPW, NPW)])
    pltpu.sync_copy(ones_hbm, ones_v)
    plsc.subcore_barrier()

    def body(i, carry):
        base = wid * EPW + i * CH_DEG
        pltpu.sync_copy(dst_hbm.at[pl.ds(base, CH_DEG)], dst_v)
        pltpu.sync_copy(ea_hbm.at[pl.ds(base, CH_DEG)], ea_v)
        pltpu.sync_copy(ea_v, acc_a.at[dst_v], add=True)
        pltpu.sync_copy(ones_v, acc_d.at[dst_v], add=True)
        return carry

    lax.fori_loop(0, EPW // CH_DEG, body, 0)
    plsc.subcore_barrier()
    pltpu.sync_copy(acc_a.at[pl.ds(s * NPW, NPW)],
                    outa_hbm.at[c, pl.ds(s * NPW, NPW)])
    pltpu.sync_copy(acc_d.at[pl.ds(s * NPW, NPW)],
                    outd_hbm.at[c, pl.ds(s * NPW, NPW)])


def _k_deg(dst, ea, ones8, z16, z8):
    return pl.kernel(
        _deg_body,
        out_type=(jax.ShapeDtypeStruct((NC, NP, ED), jnp.float32),
                  jax.ShapeDtypeStruct((NC, NP, 8), jnp.float32)),
        mesh=_mesh(),
        compiler_params=_SC_PARAMS,
        scratch_types=[pltpu.VMEM((CH_DEG,), jnp.int32),
                       pltpu.VMEM((CH_DEG, ED), jnp.float32),
                       pltpu.VMEM((CH_DEG, 8), jnp.float32),
                       pltpu.VMEM_SHARED((NP, ED), jnp.float32),
                       pltpu.VMEM_SHARED((NP, 8), jnp.float32)],
    )(dst, ea, ones8, z16, z8)


def _gather_body(ch, xl_hbm, xr_hbm, src_hbm, dst_hbm, xj_hbm, xi_hbm,
                 src_v, dst_v, rowa_v, rowb_v):
    c = lax.axis_index("c")
    s = lax.axis_index("s")
    wid = c * NS + s

    def body(i, carry):
        base = wid * EPW + i * ch
        pltpu.sync_copy(src_hbm.at[pl.ds(base, ch)], src_v)
        pltpu.sync_copy(dst_hbm.at[pl.ds(base, ch)], dst_v)
        pltpu.sync_copy(xl_hbm.at[src_v], rowa_v)
        pltpu.sync_copy(xr_hbm.at[dst_v], rowb_v)
        pltpu.sync_copy(rowa_v, xj_hbm.at[pl.ds(base, ch)])
        pltpu.sync_copy(rowb_v, xi_hbm.at[pl.ds(base, ch)])
        return carry

    lax.fori_loop(0, EPW // ch, body, 0)


def _k_gather(xl, xr, src, dst, f, ch):
    return pl.kernel(
        functools.partial(_gather_body, ch),
        out_type=(jax.ShapeDtypeStruct((E, f), jnp.float32),
                  jax.ShapeDtypeStruct((E, f), jnp.float32)),
        mesh=_mesh(),
        compiler_params=_SC_PARAMS,
        scratch_types=[pltpu.VMEM((ch,), jnp.int32),
                       pltpu.VMEM((ch,), jnp.int32),
                       pltpu.VMEM((ch, f), jnp.float32),
                       pltpu.VMEM((ch, f), jnp.float32)],
    )(xl, xr, src, dst)


def _scat1_body(p_hbm, dst_hbm, z_hbm, out_hbm, dst_v, p_v, acc):
    c = lax.axis_index("c")
    s = lax.axis_index("s")
    pltpu.sync_copy(z_hbm, acc.at[pl.ds(s * NPW, NPW)])
    plsc.subcore_barrier()

    def body(i, carry):
        base = s * EPW1 + i * CH_S1
        pltpu.sync_copy(dst_hbm.at[pl.ds(base, CH_S1)], dst_v)
        pltpu.sync_copy(p_hbm.at[c, pl.ds(base, CH_S1)], p_v)
        pltpu.sync_copy(p_v, acc.at[dst_v], add=True)
        return carry

    lax.fori_loop(0, EPW1 // CH_S1, body, 0)
    plsc.subcore_barrier()
    pltpu.sync_copy(acc.at[pl.ds(s * NPW, NPW)],
                    out_hbm.at[c, pl.ds(s * NPW, NPW)])


def _k_scat1(p, dst, z144):
    return pl.kernel(
        _scat1_body,
        out_type=jax.ShapeDtypeStruct((NC, NP, W1), jnp.float32),
        mesh=_mesh(),
        compiler_params=_SC_PARAMS,
        scratch_types=[pltpu.VMEM((CH_S1,), jnp.int32),
                       pltpu.VMEM((CH_S1, W1), jnp.float32),
                       pltpu.VMEM_SHARED((NP, W1), jnp.float32)],
    )(p, dst, z144)


def _scat2_body(p_hbm, dst_hbm, z_hbm, out_hbm, dst_v, p_v, acc):
    c = lax.axis_index("c")
    s = lax.axis_index("s")
    wid = c * NS + s
    pltpu.sync_copy(z_hbm, acc.at[pl.ds(s * NPW, NPW)])
    plsc.subcore_barrier()

    def body(i, carry):
        base = wid * EPW + i * CH_S2
        pltpu.sync_copy(dst_hbm.at[pl.ds(base, CH_S2)], dst_v)
        pltpu.sync_copy(p_hbm.at[pl.ds(base, CH_S2)], p_v)
        pltpu.sync_copy(p_v, acc.at[dst_v], add=True)
        return carry

    lax.fori_loop(0, EPW // CH_S2, body, 0)
    plsc.subcore_barrier()
    pltpu.sync_copy(acc.at[pl.ds(s * NPW, NPW)],
                    out_hbm.at[c, pl.ds(s * NPW, NPW)])


def _k_scat2(p, dst, z80):
    return pl.kernel(
        _scat2_body,
        out_type=jax.ShapeDtypeStruct((NC, NP, W2), jnp.float32),
        mesh=_mesh(),
        compiler_params=_SC_PARAMS,
        scratch_types=[pltpu.VMEM((CH_S2,), jnp.int32),
                       pltpu.VMEM((CH_S2, W2), jnp.float32),
                       pltpu.VMEM_SHARED((NP, W2), jnp.float32)],
    )(p, dst, z80)


def kernel(x, edge_index, edge_attr, lin_in_W, lin_in_b, g1_Wl, g1_bl, g1_Wr,
           g1_br, g1_We, g1_att, g1_bias, res1_W, res1_b, bn1_g, bn1_b, g2_Wl,
           g2_bl, g2_Wr, g2_br, g2_We, g2_att, g2_bias, res2_W, res2_b, bn2_g,
           bn2_b, out_W, out_b):
    src = edge_index[0]
    dst = edge_index[1]
    f32 = jnp.float32
    attv1 = g1_att.reshape(1, F1)
    attv2 = g2_att.reshape(1, HC)
    s4 = jnp.concatenate(
        [jnp.repeat(jnp.eye(H, dtype=f32), HC, axis=0),
         jnp.zeros((F1, 8 - H), f32)], axis=1)
    s4t = s4.T
    ones8 = jnp.ones((CH_DEG, 8), f32)
    z16 = jnp.zeros((NPW, ED), f32)
    z8 = jnp.zeros((NPW, 8), f32)
    z144 = jnp.zeros((NPW, W1), f32)
    z80 = jnp.zeros((NPW, W2), f32)

    h0 = _k_in(x, lin_in_W, lin_in_b.reshape(1, HC))
    outa, outd = _k_deg(dst, edge_attr, ones8, z16, z8)

    xl1, xr1 = _k_lin(h0, g1_Wl, g1_bl.reshape(1, F1), g1_Wr, g1_br.reshape(1, F1))
    xj1, xi1 = _k_gather(xl1, xr1, src, dst, F1, CH_G1)
    p1 = _k_edge1(xj1, xi1, edge_attr, g1_We, attv1, s4, s4t)
    acc1 = _k_scat1(p1, dst, z144)
    h1 = _k_post1(h0, xl1, xr1, acc1[0], acc1[1], outa[0], outa[1],
                  outd[0], outd[1], g1_We, attv1, s4, s4t,
                  g1_bias.reshape(1, F1), res1_W, res1_b.reshape(1, F1),
                  bn1_g.reshape(1, F1), bn1_b.reshape(1, F1))

    xl2, xr2 = _k_lin(h1, g2_Wl, g2_bl.reshape(1, HC), g2_Wr, g2_br.reshape(1, HC))
    xj2, xi2 = _k_gather(xl2, xr2, src, dst, HC, CH_G2)
    p2 = _k_edge2(xj2, xi2, edge_attr, g2_We, attv2)
    acc2 = _k_scat2(p2, dst, z80)
    out = _k_post2(h1, xl2, xr2, acc2[0], acc2[1], outa[0], outa[1],
                   outd[0], outd[1], g2_We, attv2,
                   g2_bias.reshape(1, HC), res2_W, res2_b.reshape(1, HC),
                   bn2_g.reshape(1, HC), bn2_b.reshape(1, HC),
                   out_W.reshape(1, HC), out_b.reshape(1, 1))
    return out


# speedup vs baseline: 20.1704x; 1.5417x over previous
import functools

import jax
import jax.numpy as jnp
from jax import lax
from jax.experimental import pallas as pl
from jax.experimental.pallas import tpu as pltpu
from jax.experimental.pallas import tpu_sc as plsc

N, E, F_IN, ED, H, HC = 10000, 320000, 128, 16, 4, 64
F1 = H * HC
W1 = 128
WD = 8
W2 = 128
NC, NS = 2, 16
NP = 10240
NPW = NP // NS
EPW = E // (NC * NS)
EPW1 = E // NS
CH_DEG, CH_G1, CH_G2, CH_S1, CH_SD, CH_S2 = 1000, 200, 400, 200, 1000, 200
RSQ = 1.0 / (1.0 + 1e-5) ** 0.5


def _mesh():
    return plsc.VectorSubcoreMesh(
        core_axis_name="c", subcore_axis_name="s",
        num_cores=NC, num_subcores=NS)


_SC_PARAMS = pltpu.CompilerParams(use_tc_tiling_on_sc=False)
_SC_PARAMS_T = pltpu.CompilerParams(use_tc_tiling_on_sc=True)


def _k_in_body(x_ref, w_ref, b_ref, o_ref):
    o_ref[...] = jax.nn.relu(
        jnp.dot(x_ref[...], w_ref[...], preferred_element_type=jnp.float32)
        + b_ref[...])


def _k_in(x, w, b):
    return pl.pallas_call(
        _k_in_body,
        out_shape=jax.ShapeDtypeStruct((N, HC), jnp.float32),
        grid=(25,),
        in_specs=[pl.BlockSpec((400, F_IN), lambda i: (i, 0)),
                  pl.BlockSpec((F_IN, HC), lambda i: (0, 0)),
                  pl.BlockSpec((1, HC), lambda i: (0, 0))],
        out_specs=pl.BlockSpec((400, HC), lambda i: (i, 0)),
    )(x, w, b)


def _k_lin_body(h_ref, wl_ref, bl_ref, wr_ref, br_ref, xl_ref, xr_ref):
    h = h_ref[...]
    xl_ref[...] = jnp.dot(h, wl_ref[...], preferred_element_type=jnp.float32) + bl_ref[...]
    xr_ref[...] = jnp.dot(h, wr_ref[...], preferred_element_type=jnp.float32) + br_ref[...]


def _k_lin(h, wl, bl, wr, br):
    fin, fout = wl.shape
    return pl.pallas_call(
        _k_lin_body,
        out_shape=(jax.ShapeDtypeStruct((N, fout), jnp.float32),
                   jax.ShapeDtypeStruct((N, fout), jnp.float32)),
        grid=(25,),
        in_specs=[pl.BlockSpec((400, fin), lambda i: (i, 0)),
                  pl.BlockSpec((fin, fout), lambda i: (0, 0)),
                  pl.BlockSpec((1, fout), lambda i: (0, 0)),
                  pl.BlockSpec((fin, fout), lambda i: (0, 0)),
                  pl.BlockSpec((1, fout), lambda i: (0, 0))],
        out_specs=(pl.BlockSpec((400, fout), lambda i: (i, 0)),
                   pl.BlockSpec((400, fout), lambda i: (i, 0))),
    )(h, wl, bl, wr, br)


def _k_lin2_body(h_ref, wl_ref, bl_ref, wr_ref, br_ref, o_ref):
    h = h_ref[...]
    xl = jnp.dot(h, wl_ref[...], preferred_element_type=jnp.float32) + bl_ref[...]
    xr = jnp.dot(h, wr_ref[...], preferred_element_type=jnp.float32) + br_ref[...]
    o_ref[...] = jnp.concatenate([xl, xr], axis=1)


def _k_lin2(h, wl, bl, wr, br):
    return pl.pallas_call(
        _k_lin2_body,
        out_shape=jax.ShapeDtypeStruct((N, 2 * HC), jnp.float32),
        grid=(25,),
        in_specs=[pl.BlockSpec((400, F1), lambda i: (i, 0)),
                  pl.BlockSpec((F1, HC), lambda i: (0, 0)),
                  pl.BlockSpec((1, HC), lambda i: (0, 0)),
                  pl.BlockSpec((F1, HC), lambda i: (0, 0)),
                  pl.BlockSpec((1, HC), lambda i: (0, 0))],
        out_specs=pl.BlockSpec((400, 2 * HC), lambda i: (i, 0)),
    )(h, wl, bl, wr, br)


def _k_edge1_body(xj_ref, xi_ref, ea_ref, we_ref, attv_ref, s4_ref, s4t_ref, o_ref, s_ref):
    xj = xj_ref[...]
    ee = jnp.dot(ea_ref[...], we_ref[...], preferred_element_type=jnp.float32)
    t = xj + xi_ref[...] + ee
    t = jnp.where(t >= 0, t, 0.2 * t)
    al = jnp.dot(t * attv_ref[...], s4_ref[...], preferred_element_type=jnp.float32)
    s = jnp.exp(al)
    pw = jnp.dot(s, s4t_ref[...], preferred_element_type=jnp.float32) * xj
    eb = pw.shape[0]
    pad = jnp.zeros((eb, WD - 2), jnp.float32)
    o_ref[0] = pw[:, 0:128]
    o_ref[1] = pw[:, 128:256]
    s_ref[0] = jnp.concatenate([s[:, 0:2], pad], axis=1)
    s_ref[1] = jnp.concatenate([s[:, 2:4], pad], axis=1)


def _k_edge1(xj, xi, ea, we, attv, s4, s4t):
    eb = 3200
    return pl.pallas_call(
        _k_edge1_body,
        out_shape=(jax.ShapeDtypeStruct((NC, E, W1), jnp.float32),
                   jax.ShapeDtypeStruct((NC, E, WD), jnp.float32)),
        grid=(E // eb,),
        in_specs=[pl.BlockSpec((eb, F1), lambda i: (i, 0)),
                  pl.BlockSpec((eb, F1), lambda i: (i, 0)),
                  pl.BlockSpec((eb, ED), lambda i: (i, 0)),
                  pl.BlockSpec((ED, F1), lambda i: (0, 0)),
                  pl.BlockSpec((1, F1), lambda i: (0, 0)),
                  pl.BlockSpec((F1, 8), lambda i: (0, 0)),
                  pl.BlockSpec((8, F1), lambda i: (0, 0))],
        out_specs=(pl.BlockSpec((NC, eb, W1), lambda i: (0, i, 0)),
                   pl.BlockSpec((NC, eb, WD), lambda i: (0, i, 0))),
    )(xj, xi, ea, we, attv, s4, s4t)


def _k_edge2_body(xjx_ref, xix_ref, ea_ref, we_ref, attv_ref, o_ref):
    xj = xjx_ref[:, 0:HC]
    xi = xix_ref[:, HC:2 * HC]
    ee = jnp.dot(ea_ref[...], we_ref[...], preferred_element_type=jnp.float32)
    t = xj + xi + ee
    t = jnp.where(t >= 0, t, 0.2 * t)
    al = jnp.sum(t * attv_ref[...], axis=1, keepdims=True)
    s = jnp.exp(al)
    pw = s * xj
    eb = pw.shape[0]
    pad = jnp.zeros((eb, W2 - HC - 1), jnp.float32)
    o_ref[...] = jnp.concatenate([pw, s, pad], axis=1)


def _k_edge2(xj, xi, ea, we, attv):
    eb = 8000
    return pl.pallas_call(
        _k_edge2_body,
        out_shape=jax.ShapeDtypeStruct((E, W2), jnp.float32),
        grid=(E // eb,),
        in_specs=[pl.BlockSpec((eb, 2 * HC), lambda i: (i, 0)),
                  pl.BlockSpec((eb, 2 * HC), lambda i: (i, 0)),
                  pl.BlockSpec((eb, ED), lambda i: (i, 0)),
                  pl.BlockSpec((ED, HC), lambda i: (0, 0)),
                  pl.BlockSpec((1, HC), lambda i: (0, 0))],
        out_specs=pl.BlockSpec((eb, W2), lambda i: (i, 0)),
    )(xj, xi, ea, we, attv)


def _loop_attr_blk(aa0, aa1, ad0, ad1):
    deg = ad0[:, 0:1] + ad1[:, 0:1]
    return (aa0 + aa1) / jnp.maximum(deg, 1.0)


def _k_post1_body(h0_ref, xl_ref, xr_ref, a0_ref, a1_ref, d0_ref, d1_ref,
                  aa0_ref, aa1_ref, ad0_ref, ad1_ref, we_ref, attv_ref,
                  s4_ref, s4t_ref, bias_ref, rw_ref, rb_ref, bg_ref, bb_ref,
                  o_ref):
    xl = xl_ref[...]
    la = _loop_attr_blk(aa0_ref[...], aa1_ref[...], ad0_ref[...], ad1_ref[...])
    eel = jnp.dot(la, we_ref[...], preferred_element_type=jnp.float32)
    t = xl + xr_ref[...] + eel
    t = jnp.where(t >= 0, t, 0.2 * t)
    al = jnp.dot(t * attv_ref[...], s4_ref[...], preferred_element_type=jnp.float32)
    sl = jnp.exp(al)
    num = jnp.concatenate([a0_ref[...], a1_ref[...]], axis=1)
    num = num + jnp.dot(sl, s4t_ref[...], preferred_element_type=jnp.float32) * xl
    nb = xl.shape[0]
    den = jnp.concatenate([d0_ref[:, 0:2], d1_ref[:, 0:2],
                           jnp.zeros((nb, 4), jnp.float32)], axis=1) + sl
    denr = jnp.dot(den, s4t_ref[...], preferred_element_type=jnp.float32)
    gat = num / denr + bias_ref[...]
    res = jnp.dot(h0_ref[...], rw_ref[...], preferred_element_type=jnp.float32) + rb_ref[...]
    o_ref[...] = jax.nn.relu(bg_ref[...] * (gat + res) * RSQ + bb_ref[...])


def _k_post1(h0, xl1, xr1, a0, a1, d0, d1, aa0, aa1, ad0, ad1, we, attv,
             s4, s4t, bias, rw, rb, bg, bb):
    nb = 400
    return pl.pallas_call(
        _k_post1_body,
        out_shape=jax.ShapeDtypeStruct((N, F1), jnp.float32),
        grid=(N // nb,),
        in_specs=[pl.BlockSpec((nb, HC), lambda i: (i, 0)),
                  pl.BlockSpec((nb, F1), lambda i: (i, 0)),
                  pl.BlockSpec((nb, F1), lambda i: (i, 0)),
                  pl.BlockSpec((nb, W1), lambda i: (i, 0)),
                  pl.BlockSpec((nb, W1), lambda i: (i, 0)),
                  pl.BlockSpec((nb, WD), lambda i: (i, 0)),
                  pl.BlockSpec((nb, WD), lambda i: (i, 0)),
                  pl.BlockSpec((nb, ED), lambda i: (i, 0)),
                  pl.BlockSpec((nb, ED), lambda i: (i, 0)),
                  pl.BlockSpec((nb, 8), lambda i: (i, 0)),
                  pl.BlockSpec((nb, 8), lambda i: (i, 0)),
                  pl.BlockSpec((ED, F1), lambda i: (0, 0)),
                  pl.BlockSpec((1, F1), lambda i: (0, 0)),
                  pl.BlockSpec((F1, 8), lambda i: (0, 0)),
                  pl.BlockSpec((8, F1), lambda i: (0, 0)),
                  pl.BlockSpec((1, F1), lambda i: (0, 0)),
                  pl.BlockSpec((HC, F1), lambda i: (0, 0)),
                  pl.BlockSpec((1, F1), lambda i: (0, 0)),
                  pl.BlockSpec((1, F1), lambda i: (0, 0)),
                  pl.BlockSpec((1, F1), lambda i: (0, 0))],
        out_specs=pl.BlockSpec((nb, F1), lambda i: (i, 0)),
    )(h0, xl1, xr1, a0, a1, d0, d1, aa0, aa1, ad0, ad1,
      we, attv, s4, s4t, bias, rw, rb, bg, bb)


def _k_post2_body(h1_ref, comb_ref, a0_ref, a1_ref, aa0_ref, aa1_ref,
                  ad0_ref, ad1_ref, we_ref, attv_ref, bias_ref, rw_ref, rb_ref,
                  bg_ref, bb_ref, ow_ref, ob_ref, o_ref):
    xl = comb_ref[:, 0:HC]
    la = _loop_attr_blk(aa0_ref[...], aa1_ref[...], ad0_ref[...], ad1_ref[...])
    eel = jnp.dot(la, we_ref[...], preferred_element_type=jnp.float32)
    t = xl + comb_ref[:, HC:2 * HC] + eel
    t = jnp.where(t >= 0, t, 0.2 * t)
    al = jnp.sum(t * attv_ref[...], axis=1, keepdims=True)
    sl = jnp.exp(al)
    a0 = a0_ref[...]
    a1 = a1_ref[...]
    num = a0[:, 0:HC] + a1[:, 0:HC] + sl * xl
    den = a0[:, HC:HC + 1] + a1[:, HC:HC + 1] + sl
    gat = num / den + bias_ref[...]
    res = jnp.dot(h1_ref[...], rw_ref[...], preferred_element_type=jnp.float32) + rb_ref[...]
    h2 = jax.nn.relu(bg_ref[...] * (gat + res) * RSQ + bb_ref[...])
    o_ref[...] = jnp.sum(h2 * ow_ref[...], axis=1, keepdims=True) + ob_ref[...]


def _k_post2(h1, comb, a0, a1, aa0, aa1, ad0, ad1, we, attv, bias,
             rw, rb, bg, bb, ow, ob):
    nb = 400
    return pl.pallas_call(
        _k_post2_body,
        out_shape=jax.ShapeDtypeStruct((N, 1), jnp.float32),
        grid=(N // nb,),
        in_specs=[pl.BlockSpec((nb, F1), lambda i: (i, 0)),
                  pl.BlockSpec((nb, 2 * HC), lambda i: (i, 0)),
                  pl.BlockSpec((nb, W2), lambda i: (i, 0)),
                  pl.BlockSpec((nb, W2), lambda i: (i, 0)),
                  pl.BlockSpec((nb, ED), lambda i: (i, 0)),
                  pl.BlockSpec((nb, ED), lambda i: (i, 0)),
                  pl.BlockSpec((nb, 8), lambda i: (i, 0)),
                  pl.BlockSpec((nb, 8), lambda i: (i, 0)),
                  pl.BlockSpec((ED, HC), lambda i: (0, 0)),
                  pl.BlockSpec((1, HC), lambda i: (0, 0)),
                  pl.BlockSpec((1, HC), lambda i: (0, 0)),
                  pl.BlockSpec((F1, HC), lambda i: (0, 0)),
                  pl.BlockSpec((1, HC), lambda i: (0, 0)),
                  pl.BlockSpec((1, HC), lambda i: (0, 0)),
                  pl.BlockSpec((1, HC), lambda i: (0, 0)),
                  pl.BlockSpec((1, HC), lambda i: (0, 0)),
                  pl.BlockSpec((1, 1), lambda i: (0, 0))],
        out_specs=pl.BlockSpec((nb, 1), lambda i: (i, 0)),
    )(h1, comb, a0, a1, aa0, aa1, ad0, ad1,
      we, attv, bias, rw, rb, bg, bb, ow, ob)


def _deg_body(dst_hbm, ea_hbm, ones_hbm, z16_hbm, z8_hbm, outa_hbm, outd_hbm,
              dst_v, ea_v, ones_v, acc_a, acc_d):
    c = lax.axis_index("c")
    s = lax.axis_index("s")
    wid = c * NS + s
    pltpu.sync_copy(z16_hbm, acc_a.at[pl.ds(s * NPW, NPW)])
    pltpu.sync_copy(z8_hbm, acc_d.at[pl.ds(s * NPW, NPW)])
    pltpu.sync_copy(ones_hbm, ones_v)
    plsc.subcore_barrier()

    def body(i, carry):
        base = wid * EPW + i * CH_DEG
        pltpu.sync_copy(dst_hbm.at[pl.ds(base, CH_DEG)], dst_v)
        pltpu.sync_copy(ea_hbm.at[pl.ds(base, CH_DEG)], ea_v)
        pltpu.sync_copy(ea_v, acc_a.at[dst_v], add=True)
        pltpu.sync_copy(ones_v, acc_d.at[dst_v], add=True)
        return carry

    lax.fori_loop(0, EPW // CH_DEG, body, 0)
    plsc.subcore_barrier()
    pltpu.sync_copy(acc_a.at[pl.ds(s * NPW, NPW)],
                    outa_hbm.at[c, pl.ds(s * NPW, NPW)])
    pltpu.sync_copy(acc_d.at[pl.ds(s * NPW, NPW)],
                    outd_hbm.at[c, pl.ds(s * NPW, NPW)])


def _k_deg(dst, ea, ones8, z16, z8):
    return pl.kernel(
        _deg_body,
        out_type=(jax.ShapeDtypeStruct((NC, NP, ED), jnp.float32),
                  jax.ShapeDtypeStruct((NC, NP, 8), jnp.float32)),
        mesh=_mesh(),
        compiler_params=_SC_PARAMS,
        scratch_types=[pltpu.VMEM((CH_DEG,), jnp.int32),
                       pltpu.VMEM((CH_DEG, ED), jnp.float32),
                       pltpu.VMEM((CH_DEG, 8), jnp.float32),
                       pltpu.VMEM_SHARED((NP, ED), jnp.float32),
                       pltpu.VMEM_SHARED((NP, 8), jnp.float32)],
    )(dst, ea, ones8, z16, z8)


def _gather_body(ch, xl_hbm, xr_hbm, src_hbm, dst_hbm, xj_hbm, xi_hbm,
                 src_v, dst_v, rowa_v, rowb_v):
    c = lax.axis_index("c")
    s = lax.axis_index("s")
    wid = c * NS + s

    def body(i, carry):
        base = wid * EPW + i * ch
        pltpu.sync_copy(src_hbm.at[pl.ds(base, ch)], src_v)
        pltpu.sync_copy(dst_hbm.at[pl.ds(base, ch)], dst_v)
        pltpu.sync_copy(xl_hbm.at[src_v], rowa_v)
        pltpu.sync_copy(xr_hbm.at[dst_v], rowb_v)
        pltpu.sync_copy(rowa_v, xj_hbm.at[pl.ds(base, ch)])
        pltpu.sync_copy(rowb_v, xi_hbm.at[pl.ds(base, ch)])
        return carry

    lax.fori_loop(0, EPW // ch, body, 0)


def _k_gather(xl, xr, src, dst, f, ch):
    return pl.kernel(
        functools.partial(_gather_body, ch),
        out_type=(jax.ShapeDtypeStruct((E, f), jnp.float32),
                  jax.ShapeDtypeStruct((E, f), jnp.float32)),
        mesh=_mesh(),
        compiler_params=_SC_PARAMS_T,
        scratch_types=[pltpu.VMEM((ch,), jnp.int32),
                       pltpu.VMEM((ch,), jnp.int32),
                       pltpu.VMEM((ch, f), jnp.float32),
                       pltpu.VMEM((ch, f), jnp.float32)],
    )(xl, xr, src, dst)


def _scat1_body(p_hbm, dst_hbm, z_hbm, out_hbm, dst_v, p_v, acc):
    c = lax.axis_index("c")
    s = lax.axis_index("s")
    pltpu.sync_copy(z_hbm, acc.at[pl.ds(s * NPW, NPW)])
    plsc.subcore_barrier()

    def body(i, carry):
        base = s * EPW1 + i * CH_S1
        pltpu.sync_copy(dst_hbm.at[pl.ds(base, CH_S1)], dst_v)
        pltpu.sync_copy(p_hbm.at[c, pl.ds(base, CH_S1)], p_v)
        pltpu.sync_copy(p_v, acc.at[dst_v], add=True)
        return carry

    lax.fori_loop(0, EPW1 // CH_S1, body, 0)
    plsc.subcore_barrier()
    pltpu.sync_copy(acc.at[pl.ds(s * NPW, NPW)],
                    out_hbm.at[c, pl.ds(s * NPW, NPW)])


def _k_scat1(p, dst, z128):
    return pl.kernel(
        _scat1_body,
        out_type=jax.ShapeDtypeStruct((NC, NP, W1), jnp.float32),
        mesh=_mesh(),
        compiler_params=_SC_PARAMS_T,
        scratch_types=[pltpu.VMEM((CH_S1,), jnp.int32),
                       pltpu.VMEM((CH_S1, W1), jnp.float32),
                       pltpu.VMEM_SHARED((NP, W1), jnp.float32)],
    )(p, dst, z128)


def _scatd_body(p_hbm, dst_hbm, z_hbm, out_hbm, dst_v, p_v, acc):
    c = lax.axis_index("c")
    s = lax.axis_index("s")
    pltpu.sync_copy(z_hbm, acc.at[pl.ds(s * NPW, NPW)])
    plsc.subcore_barrier()

    def body(i, carry):
        base = s * EPW1 + i * CH_SD
        pltpu.sync_copy(dst_hbm.at[pl.ds(base, CH_SD)], dst_v)
        pltpu.sync_copy(p_hbm.at[c, pl.ds(base, CH_SD)], p_v)
        pltpu.sync_copy(p_v, acc.at[dst_v], add=True)
        return carry

    lax.fori_loop(0, EPW1 // CH_SD, body, 0)
    plsc.subcore_barrier()
    pltpu.sync_copy(acc.at[pl.ds(s * NPW, NPW)],
                    out_hbm.at[c, pl.ds(s * NPW, NPW)])


def _k_scatd(p, dst, z8):
    return pl.kernel(
        _scatd_body,
        out_type=jax.ShapeDtypeStruct((NC, NP, WD), jnp.float32),
        mesh=_mesh(),
        compiler_params=_SC_PARAMS,
        scratch_types=[pltpu.VMEM((CH_SD,), jnp.int32),
                       pltpu.VMEM((CH_SD, WD), jnp.float32),
                       pltpu.VMEM_SHARED((NP, WD), jnp.float32)],
    )(p, dst, z8)


def _scat2_body(p_hbm, dst_hbm, z_hbm, out_hbm, dst_v, p_v, acc):
    c = lax.axis_index("c")
    s = lax.axis_index("s")
    wid = c * NS + s
    pltpu.sync_copy(z_hbm, acc.at[pl.ds(s * NPW, NPW)])
    plsc.subcore_barrier()

    def body(i, carry):
        base = wid * EPW + i * CH_S2
        pltpu.sync_copy(dst_hbm.at[pl.ds(base, CH_S2)], dst_v)
        pltpu.sync_copy(p_hbm.at[pl.ds(base, CH_S2)], p_v)
        pltpu.sync_copy(p_v, acc.at[dst_v], add=True)
        return carry

    lax.fori_loop(0, EPW // CH_S2, body, 0)
    plsc.subcore_barrier()
    pltpu.sync_copy(acc.at[pl.ds(s * NPW, NPW)],
                    out_hbm.at[c, pl.ds(s * NPW, NPW)])


def _k_scat2(p, dst, z128):
    return pl.kernel(
        _scat2_body,
        out_type=jax.ShapeDtypeStruct((NC, NP, W2), jnp.float32),
        mesh=_mesh(),
        compiler_params=_SC_PARAMS_T,
        scratch_types=[pltpu.VMEM((CH_S2,), jnp.int32),
                       pltpu.VMEM((CH_S2, W2), jnp.float32),
                       pltpu.VMEM_SHARED((NP, W2), jnp.float32)],
    )(p, dst, z128)


def kernel(x, edge_index, edge_attr, lin_in_W, lin_in_b, g1_Wl, g1_bl, g1_Wr,
           g1_br, g1_We, g1_att, g1_bias, res1_W, res1_b, bn1_g, bn1_b, g2_Wl,
           g2_bl, g2_Wr, g2_br, g2_We, g2_att, g2_bias, res2_W, res2_b, bn2_g,
           bn2_b, out_W, out_b):
    src = edge_index[0]
    dst = edge_index[1]
    f32 = jnp.float32
    attv1 = g1_att.reshape(1, F1)
    attv2 = g2_att.reshape(1, HC)
    s4 = jnp.concatenate(
        [jnp.repeat(jnp.eye(H, dtype=f32), HC, axis=0),
         jnp.zeros((F1, 8 - H), f32)], axis=1)
    s4t = s4.T
    ones8 = jnp.ones((CH_DEG, 8), f32)
    z16 = jnp.zeros((NPW, ED), f32)
    z8 = jnp.zeros((NPW, 8), f32)
    z128 = jnp.zeros((NPW, 128), f32)
    zd = jnp.zeros((NPW, WD), f32)

    h0 = _k_in(x, lin_in_W, lin_in_b.reshape(1, HC))
    outa, outd = _k_deg(dst, edge_attr, ones8, z16, z8)

    xl1, xr1 = _k_lin(h0, g1_Wl, g1_bl.reshape(1, F1), g1_Wr, g1_br.reshape(1, F1))
    xj1, xi1 = _k_gather(xl1, xr1, src, dst, F1, CH_G1)
    p1, s1 = _k_edge1(xj1, xi1, edge_attr, g1_We, attv1, s4, s4t)
    acc1 = _k_scat1(p1, dst, z128)
    den1 = _k_scatd(s1, dst, zd)
    h1 = _k_post1(h0, xl1, xr1, acc1[0], acc1[1], den1[0], den1[1],
                  outa[0], outa[1], outd[0], outd[1], g1_We, attv1, s4, s4t,
                  g1_bias.reshape(1, F1), res1_W, res1_b.reshape(1, F1),
                  bn1_g.reshape(1, F1), bn1_b.reshape(1, F1))

    comb = _k_lin2(h1, g2_Wl, g2_bl.reshape(1, HC), g2_Wr, g2_br.reshape(1, HC))
    xjx, xix = _k_gather(comb, comb, src, dst, 2 * HC, CH_G2)
    p2 = _k_edge2(xjx, xix, edge_attr, g2_We, attv2)
    acc2 = _k_scat2(p2, dst, z128)
    out = _k_post2(h1, comb, acc2[0], acc2[1], outa[0], outa[1],
                   outd[0], outd[1], g2_We, attv2,
                   g2_bias.reshape(1, HC), res2_W, res2_b.reshape(1, HC),
                   bn2_g.reshape(1, HC), bn2_b.reshape(1, HC),
                   out_W.reshape(1, HC), out_b.reshape(1, 1))
    return out


# speedup vs baseline: 22.1923x; 1.1002x over previous
import functools

import jax
import jax.numpy as jnp
from jax import lax
from jax.experimental import pallas as pl
from jax.experimental.pallas import tpu as pltpu
from jax.experimental.pallas import tpu_sc as plsc

N, E, F_IN, ED, H, HC = 10000, 320000, 128, 16, 4, 64
F1 = H * HC
W1 = 128
WD = 8
W2 = 128
NC, NS = 2, 16
NP = 10240
NPW = NP // NS
EPW = E // (NC * NS)
EPW1 = E // NS
CH_DEG, CH_G1, CH_G2, CH_S1, CH_SD, CH_S2 = 1000, 200, 400, 200, 1000, 200
RSQ = 1.0 / (1.0 + 1e-5) ** 0.5


def _mesh():
    return plsc.VectorSubcoreMesh(
        core_axis_name="c", subcore_axis_name="s",
        num_cores=NC, num_subcores=NS)


_SC_PARAMS = pltpu.CompilerParams(use_tc_tiling_on_sc=False,
                                  needs_layout_passes=False)
_SC_PARAMS_T = pltpu.CompilerParams(use_tc_tiling_on_sc=True)


def _k_in_body(x_ref, w_ref, b_ref, o_ref):
    o_ref[...] = jax.nn.relu(
        jnp.dot(x_ref[...], w_ref[...], preferred_element_type=jnp.float32)
        + b_ref[...])


def _k_in(x, w, b):
    return pl.pallas_call(
        _k_in_body,
        out_shape=jax.ShapeDtypeStruct((N, HC), jnp.float32),
        grid=(25,),
        in_specs=[pl.BlockSpec((400, F_IN), lambda i: (i, 0)),
                  pl.BlockSpec((F_IN, HC), lambda i: (0, 0)),
                  pl.BlockSpec((1, HC), lambda i: (0, 0))],
        out_specs=pl.BlockSpec((400, HC), lambda i: (i, 0)),
    )(x, w, b)


def _k_lin_body(h_ref, wl_ref, bl_ref, wr_ref, br_ref, xl_ref, xr_ref):
    h = h_ref[...]
    xl_ref[...] = jnp.dot(h, wl_ref[...], preferred_element_type=jnp.float32) + bl_ref[...]
    xr_ref[...] = jnp.dot(h, wr_ref[...], preferred_element_type=jnp.float32) + br_ref[...]


def _k_lin(h, wl, bl, wr, br):
    fin, fout = wl.shape
    return pl.pallas_call(
        _k_lin_body,
        out_shape=(jax.ShapeDtypeStruct((N, fout), jnp.float32),
                   jax.ShapeDtypeStruct((N, fout), jnp.float32)),
        grid=(25,),
        in_specs=[pl.BlockSpec((400, fin), lambda i: (i, 0)),
                  pl.BlockSpec((fin, fout), lambda i: (0, 0)),
                  pl.BlockSpec((1, fout), lambda i: (0, 0)),
                  pl.BlockSpec((fin, fout), lambda i: (0, 0)),
                  pl.BlockSpec((1, fout), lambda i: (0, 0))],
        out_specs=(pl.BlockSpec((400, fout), lambda i: (i, 0)),
                   pl.BlockSpec((400, fout), lambda i: (i, 0))),
    )(h, wl, bl, wr, br)


def _k_lin2_body(h_ref, wl_ref, bl_ref, wr_ref, br_ref, o_ref):
    h = h_ref[...]
    xl = jnp.dot(h, wl_ref[...], preferred_element_type=jnp.float32) + bl_ref[...]
    xr = jnp.dot(h, wr_ref[...], preferred_element_type=jnp.float32) + br_ref[...]
    o_ref[...] = jnp.concatenate([xl, xr], axis=1)


def _k_lin2(h, wl, bl, wr, br):
    return pl.pallas_call(
        _k_lin2_body,
        out_shape=jax.ShapeDtypeStruct((N, 2 * HC), jnp.float32),
        grid=(25,),
        in_specs=[pl.BlockSpec((400, F1), lambda i: (i, 0)),
                  pl.BlockSpec((F1, HC), lambda i: (0, 0)),
                  pl.BlockSpec((1, HC), lambda i: (0, 0)),
                  pl.BlockSpec((F1, HC), lambda i: (0, 0)),
                  pl.BlockSpec((1, HC), lambda i: (0, 0))],
        out_specs=pl.BlockSpec((400, 2 * HC), lambda i: (i, 0)),
    )(h, wl, bl, wr, br)


def _k_edge1_body(xj_ref, xi_ref, ea_ref, we_ref, attv_ref, s4_ref, s4t_ref, o_ref, s_ref):
    xj = xj_ref[...]
    ee = jnp.dot(ea_ref[...], we_ref[...], preferred_element_type=jnp.float32)
    t = xj + xi_ref[...] + ee
    t = jnp.where(t >= 0, t, 0.2 * t)
    ta = t * attv_ref[...]
    al = jnp.dot(ta, s4_ref[...], preferred_element_type=jnp.float32)
    s = jnp.exp(al)
    pw = jnp.dot(s, s4t_ref[...], preferred_element_type=jnp.float32) * xj
    o_ref[0] = pw[:, 0:128]
    o_ref[1] = pw[:, 128:256]
    dn = (((0,), (1,)), ((), ()))
    s_ref[0] = jnp.exp(lax.dot_general(
        s4_ref[...], ta[0:1280], dn, preferred_element_type=jnp.float32))
    s_ref[1] = jnp.exp(lax.dot_general(
        s4_ref[...], ta[1280:2560], dn, preferred_element_type=jnp.float32))


def _k_edge1(xj, xi, ea, we, attv, s4, s4t):
    eb = 2560
    return pl.pallas_call(
        _k_edge1_body,
        out_shape=(jax.ShapeDtypeStruct((NC, E, W1), jnp.float32),
                   jax.ShapeDtypeStruct((E // 1280, 8, 1280), jnp.float32)),
        grid=(E // eb,),
        in_specs=[pl.BlockSpec((eb, F1), lambda i: (i, 0)),
                  pl.BlockSpec((eb, F1), lambda i: (i, 0)),
                  pl.BlockSpec((eb, ED), lambda i: (i, 0)),
                  pl.BlockSpec((ED, F1), lambda i: (0, 0)),
                  pl.BlockSpec((1, F1), lambda i: (0, 0)),
                  pl.BlockSpec((F1, 8), lambda i: (0, 0)),
                  pl.BlockSpec((8, F1), lambda i: (0, 0))],
        out_specs=(pl.BlockSpec((NC, eb, W1), lambda i: (0, i, 0)),
                   pl.BlockSpec((2, 8, 1280), lambda i: (i, 0, 0))),
    )(xj, xi, ea, we, attv, s4, s4t)


def _k_edge2_body(xjx_ref, xix_ref, ea_ref, we_ref, attv_ref, o_ref):
    xj = xjx_ref[:, 0:HC]
    xi = xix_ref[:, HC:2 * HC]
    ee = jnp.dot(ea_ref[...], we_ref[...], preferred_element_type=jnp.float32)
    t = xj + xi + ee
    t = jnp.where(t >= 0, t, 0.2 * t)
    al = jnp.sum(t * attv_ref[...], axis=1, keepdims=True)
    s = jnp.exp(al)
    pw = s * xj
    eb = pw.shape[0]
    pad = jnp.zeros((eb, W2 - HC - 1), jnp.float32)
    o_ref[...] = jnp.concatenate([pw, s, pad], axis=1)


def _k_edge2(xj, xi, ea, we, attv):
    eb = 8000
    return pl.pallas_call(
        _k_edge2_body,
        out_shape=jax.ShapeDtypeStruct((E, W2), jnp.float32),
        grid=(E // eb,),
        in_specs=[pl.BlockSpec((eb, 2 * HC), lambda i: (i, 0)),
                  pl.BlockSpec((eb, 2 * HC), lambda i: (i, 0)),
                  pl.BlockSpec((eb, ED), lambda i: (i, 0)),
                  pl.BlockSpec((ED, HC), lambda i: (0, 0)),
                  pl.BlockSpec((1, HC), lambda i: (0, 0))],
        out_specs=pl.BlockSpec((eb, W2), lambda i: (i, 0)),
    )(xj, xi, ea, we, attv)


def _loop_attr_blk(aa0, aa1, ad0, ad1):
    deg = ad0[:, 0:1] + ad1[:, 0:1]
    return (aa0 + aa1) / jnp.maximum(deg, 1.0)


def _k_post1_body(h0_ref, xl_ref, xr_ref, a0_ref, a1_ref, dsum_ref,
                  aa0_ref, aa1_ref, ad0_ref, ad1_ref, we_ref, attv_ref,
                  s4_ref, s4t_ref, bias_ref, rw_ref, rb_ref,
                  bg_ref, bb_ref, o_ref):
    xl = xl_ref[...]
    la = _loop_attr_blk(aa0_ref[...], aa1_ref[...], ad0_ref[...], ad1_ref[...])
    eel = jnp.dot(la, we_ref[...], preferred_element_type=jnp.float32)
    t = xl + xr_ref[...] + eel
    t = jnp.where(t >= 0, t, 0.2 * t)
    al = jnp.dot(t * attv_ref[...], s4_ref[...], preferred_element_type=jnp.float32)
    sl = jnp.exp(al)
    num = jnp.concatenate([a0_ref[...], a1_ref[...]], axis=1)
    num = num + jnp.dot(sl, s4t_ref[...], preferred_element_type=jnp.float32) * xl
    dtot = jnp.sum(dsum_ref[...], axis=0) + sl
    denr = jnp.dot(dtot, s4t_ref[...], preferred_element_type=jnp.float32)
    gat = num / denr + bias_ref[...]
    res = jnp.dot(h0_ref[...], rw_ref[...], preferred_element_type=jnp.float32) + rb_ref[...]
    o_ref[...] = jax.nn.relu(bg_ref[...] * (gat + res) * RSQ + bb_ref[...])


def _k_post1(h0, xl1, xr1, a0, a1, dsum, aa0, aa1, ad0, ad1, we, attv,
             s4, s4t, bias, rw, rb, bg, bb):
    nb = 400
    return pl.pallas_call(
        _k_post1_body,
        out_shape=jax.ShapeDtypeStruct((N, F1), jnp.float32),
        grid=(N // nb,),
        in_specs=[pl.BlockSpec((nb, HC), lambda i: (i, 0)),
                  pl.BlockSpec((nb, F1), lambda i: (i, 0)),
                  pl.BlockSpec((nb, F1), lambda i: (i, 0)),
                  pl.BlockSpec((nb, W1), lambda i: (i, 0)),
                  pl.BlockSpec((nb, W1), lambda i: (i, 0)),
                  pl.BlockSpec((NC * NS, nb, 8), lambda i: (0, i, 0)),
                  pl.BlockSpec((nb, ED), lambda i: (i, 0)),
                  pl.BlockSpec((nb, ED), lambda i: (i, 0)),
                  pl.BlockSpec((nb, 8), lambda i: (i, 0)),
                  pl.BlockSpec((nb, 8), lambda i: (i, 0)),
                  pl.BlockSpec((ED, F1), lambda i: (0, 0)),
                  pl.BlockSpec((1, F1), lambda i: (0, 0)),
                  pl.BlockSpec((F1, 8), lambda i: (0, 0)),
                  pl.BlockSpec((8, F1), lambda i: (0, 0)),
                  pl.BlockSpec((1, F1), lambda i: (0, 0)),
                  pl.BlockSpec((HC, F1), lambda i: (0, 0)),
                  pl.BlockSpec((1, F1), lambda i: (0, 0)),
                  pl.BlockSpec((1, F1), lambda i: (0, 0)),
                  pl.BlockSpec((1, F1), lambda i: (0, 0))],
        out_specs=pl.BlockSpec((nb, F1), lambda i: (i, 0)),
    )(h0, xl1, xr1, a0, a1, dsum, aa0, aa1, ad0, ad1,
      we, attv, s4, s4t, bias, rw, rb, bg, bb)


def _k_post2_body(h1_ref, comb_ref, a0_ref, a1_ref, aa0_ref, aa1_ref,
                  ad0_ref, ad1_ref, we_ref, attv_ref, bias_ref, rw_ref, rb_ref,
                  bg_ref, bb_ref, ow_ref, ob_ref, o_ref):
    xl = comb_ref[:, 0:HC]
    la = _loop_attr_blk(aa0_ref[...], aa1_ref[...], ad0_ref[...], ad1_ref[...])
    eel = jnp.dot(la, we_ref[...], preferred_element_type=jnp.float32)
    t = xl + comb_ref[:, HC:2 * HC] + eel
    t = jnp.where(t >= 0, t, 0.2 * t)
    al = jnp.sum(t * attv_ref[...], axis=1, keepdims=True)
    sl = jnp.exp(al)
    a0 = a0_ref[...]
    a1 = a1_ref[...]
    num = a0[:, 0:HC] + a1[:, 0:HC] + sl * xl
    den = a0[:, HC:HC + 1] + a1[:, HC:HC + 1] + sl
    gat = num / den + bias_ref[...]
    res = jnp.dot(h1_ref[...], rw_ref[...], preferred_element_type=jnp.float32) + rb_ref[...]
    h2 = jax.nn.relu(bg_ref[...] * (gat + res) * RSQ + bb_ref[...])
    o_ref[...] = jnp.sum(h2 * ow_ref[...], axis=1, keepdims=True) + ob_ref[...]


def _k_post2(h1, comb, a0, a1, aa0, aa1, ad0, ad1, we, attv, bias,
             rw, rb, bg, bb, ow, ob):
    nb = 400
    return pl.pallas_call(
        _k_post2_body,
        out_shape=jax.ShapeDtypeStruct((N, 1), jnp.float32),
        grid=(N // nb,),
        in_specs=[pl.BlockSpec((nb, F1), lambda i: (i, 0)),
                  pl.BlockSpec((nb, 2 * HC), lambda i: (i, 0)),
                  pl.BlockSpec((nb, W2), lambda i: (i, 0)),
                  pl.BlockSpec((nb, W2), lambda i: (i, 0)),
                  pl.BlockSpec((nb, ED), lambda i: (i, 0)),
                  pl.BlockSpec((nb, ED), lambda i: (i, 0)),
                  pl.BlockSpec((nb, 8), lambda i: (i, 0)),
                  pl.BlockSpec((nb, 8), lambda i: (i, 0)),
                  pl.BlockSpec((ED, HC), lambda i: (0, 0)),
                  pl.BlockSpec((1, HC), lambda i: (0, 0)),
                  pl.BlockSpec((1, HC), lambda i: (0, 0)),
                  pl.BlockSpec((F1, HC), lambda i: (0, 0)),
                  pl.BlockSpec((1, HC), lambda i: (0, 0)),
                  pl.BlockSpec((1, HC), lambda i: (0, 0)),
                  pl.BlockSpec((1, HC), lambda i: (0, 0)),
                  pl.BlockSpec((1, HC), lambda i: (0, 0)),
                  pl.BlockSpec((1, 1), lambda i: (0, 0))],
        out_specs=pl.BlockSpec((nb, 1), lambda i: (i, 0)),
    )(h1, comb, a0, a1, aa0, aa1, ad0, ad1,
      we, attv, bias, rw, rb, bg, bb, ow, ob)


def _deg_body(dst_hbm, ea_hbm, ones_hbm, z16_hbm, z8_hbm, outa_hbm, outd_hbm,
              dst_v, ea_v, ones_v, acc_a, acc_d):
    c = lax.axis_index("c")
    s = lax.axis_index("s")
    wid = c * NS + s
    pltpu.sync_copy(z16_hbm, acc_a.at[pl.ds(s * NPW, NPW)])
    pltpu.sync_copy(z8_hbm, acc_d.at[pl.ds(s * NPW, NPW)])
    pltpu.sync_copy(ones_hbm, ones_v)
    plsc.subcore_barrier()

    def body(i, carry):
        base = wid * EPW + i * CH_DEG
        pltpu.sync_copy(dst_hbm.at[pl.ds(base, CH_DEG)], dst_v)
        pltpu.sync_copy(ea_hbm.at[pl.ds(base, CH_DEG)], ea_v)
        pltpu.sync_copy(ea_v, acc_a.at[dst_v], add=True)
        pltpu.sync_copy(ones_v, acc_d.at[dst_v], add=True)
        return carry

    lax.fori_loop(0, EPW // CH_DEG, body, 0)
    plsc.subcore_barrier()
    pltpu.sync_copy(acc_a.at[pl.ds(s * NPW, NPW)],
                    outa_hbm.at[c, pl.ds(s * NPW, NPW)])
    pltpu.sync_copy(acc_d.at[pl.ds(s * NPW, NPW)],
                    outd_hbm.at[c, pl.ds(s * NPW, NPW)])


def _k_deg(dst, ea, ones8, z16, z8):
    return pl.kernel(
        _deg_body,
        out_type=(jax.ShapeDtypeStruct((NC, NP, ED), jnp.float32),
                  jax.ShapeDtypeStruct((NC, NP, 8), jnp.float32)),
        mesh=_mesh(),
        compiler_params=_SC_PARAMS,
        scratch_types=[pltpu.VMEM((CH_DEG,), jnp.int32),
                       pltpu.VMEM((CH_DEG, ED), jnp.float32),
                       pltpu.VMEM((CH_DEG, 8), jnp.float32),
                       pltpu.VMEM_SHARED((NP, ED), jnp.float32),
                       pltpu.VMEM_SHARED((NP, 8), jnp.float32)],
    )(dst, ea, ones8, z16, z8)


def _gather_body(ch, xl_hbm, xr_hbm, src_hbm, dst_hbm, xj_hbm, xi_hbm,
                 src_v, dst_v, rowa_v, rowb_v):
    c = lax.axis_index("c")
    s = lax.axis_index("s")
    wid = c * NS + s

    def body(i, carry):
        base = wid * EPW + i * ch
        pltpu.sync_copy(src_hbm.at[pl.ds(base, ch)], src_v)
        pltpu.sync_copy(dst_hbm.at[pl.ds(base, ch)], dst_v)
        pltpu.sync_copy(xl_hbm.at[src_v], rowa_v)
        pltpu.sync_copy(xr_hbm.at[dst_v], rowb_v)
        pltpu.sync_copy(rowa_v, xj_hbm.at[pl.ds(base, ch)])
        pltpu.sync_copy(rowb_v, xi_hbm.at[pl.ds(base, ch)])
        return carry

    lax.fori_loop(0, EPW // ch, body, 0)


def _k_gather(xl, xr, src, dst, f, ch):
    return pl.kernel(
        functools.partial(_gather_body, ch),
        out_type=(jax.ShapeDtypeStruct((E, f), jnp.float32),
                  jax.ShapeDtypeStruct((E, f), jnp.float32)),
        mesh=_mesh(),
        compiler_params=_SC_PARAMS_T,
        scratch_types=[pltpu.VMEM((ch,), jnp.int32),
                       pltpu.VMEM((ch,), jnp.int32),
                       pltpu.VMEM((ch, f), jnp.float32),
                       pltpu.VMEM((ch, f), jnp.float32)],
    )(xl, xr, src, dst)


def _scat1_body(p_hbm, dst_hbm, z_hbm, out_hbm, dst_v, p_v, acc):
    c = lax.axis_index("c")
    s = lax.axis_index("s")
    pltpu.sync_copy(z_hbm, acc.at[pl.ds(s * NPW, NPW)])
    plsc.subcore_barrier()

    def body(i, carry):
        base = s * EPW1 + i * CH_S1
        pltpu.sync_copy(dst_hbm.at[pl.ds(base, CH_S1)], dst_v)
        pltpu.sync_copy(p_hbm.at[c, pl.ds(base, CH_S1)], p_v)
        pltpu.sync_copy(p_v, acc.at[dst_v], add=True)
        return carry

    lax.fori_loop(0, EPW1 // CH_S1, body, 0)
    plsc.subcore_barrier()
    pltpu.sync_copy(acc.at[pl.ds(s * NPW, NPW)],
                    out_hbm.at[c, pl.ds(s * NPW, NPW)])


def _k_scat1(p, dst, z128):
    return pl.kernel(
        _scat1_body,
        out_type=jax.ShapeDtypeStruct((NC, NP, W1), jnp.float32),
        mesh=_mesh(),
        compiler_params=_SC_PARAMS_T,
        scratch_types=[pltpu.VMEM((CH_S1,), jnp.int32),
                       pltpu.VMEM((CH_S1, W1), jnp.float32),
                       pltpu.VMEM_SHARED((NP, W1), jnp.float32)],
    )(p, dst, z128)


def _scatd_body(spk_hbm, dst_hbm, z_hbm, out_hbm, dst_v, s_stage, acc2d):
    c = lax.axis_index("c")
    s = lax.axis_index("s")
    wid = c * NS + s
    pltpu.sync_copy(z_hbm, acc2d)
    nblk = jnp.where(wid < 26, 8, 7)
    b0 = jnp.where(wid < 26, 8 * wid, 208 + 7 * (wid - 26))
    lanes = lax.broadcasted_iota(jnp.int32, (16,), 0)

    def blk(i, carry):
        b = b0 + i
        pltpu.sync_copy(spk_hbm.at[b], s_stage)
        pltpu.sync_copy(dst_hbm.at[pl.ds(b * 1280, 1280)], dst_v)

        def row(r, carry2):
            dstv = dst_v[pl.ds(r * 16, 16)]
            for h in range(H):
                sv = plsc.load_gather(
                    s_stage, [jnp.full((16,), h, jnp.int32), r * 16 + lanes])
                plsc.addupdate_scatter(
                    acc2d, [dstv, jnp.full((16,), h, jnp.int32)], sv)
            return carry2

        lax.fori_loop(0, 80, row, 0)
        return carry

    lax.fori_loop(0, nblk, blk, 0)
    pltpu.sync_copy(acc2d, out_hbm.at[wid])


def _k_scatd(spk, dst, z):
    return pl.kernel(
        _scatd_body,
        out_type=jax.ShapeDtypeStruct((NC * NS, NP, 8), jnp.float32),
        mesh=_mesh(),
        compiler_params=_SC_PARAMS,
        scratch_types=[pltpu.VMEM((1280,), jnp.int32),
                       pltpu.VMEM((8, 1280), jnp.float32),
                       pltpu.VMEM((NP, 8), jnp.float32)],
    )(spk, dst, z)


def _scat2_body(p_hbm, dst_hbm, z_hbm, out_hbm, dst_v, p_v, acc):
    c = lax.axis_index("c")
    s = lax.axis_index("s")
    wid = c * NS + s
    pltpu.sync_copy(z_hbm, acc.at[pl.ds(s * NPW, NPW)])
    plsc.subcore_barrier()

    def body(i, carry):
        base = wid * EPW + i * CH_S2
        pltpu.sync_copy(dst_hbm.at[pl.ds(base, CH_S2)], dst_v)
        pltpu.sync_copy(p_hbm.at[pl.ds(base, CH_S2)], p_v)
        pltpu.sync_copy(p_v, acc.at[dst_v], add=True)
        return carry

    lax.fori_loop(0, EPW // CH_S2, body, 0)
    plsc.subcore_barrier()
    pltpu.sync_copy(acc.at[pl.ds(s * NPW, NPW)],
                    out_hbm.at[c, pl.ds(s * NPW, NPW)])


def _k_scat2(p, dst, z128):
    return pl.kernel(
        _scat2_body,
        out_type=jax.ShapeDtypeStruct((NC, NP, W2), jnp.float32),
        mesh=_mesh(),
        compiler_params=_SC_PARAMS_T,
        scratch_types=[pltpu.VMEM((CH_S2,), jnp.int32),
                       pltpu.VMEM((CH_S2, W2), jnp.float32),
                       pltpu.VMEM_SHARED((NP, W2), jnp.float32)],
    )(p, dst, z128)


def kernel(x, edge_index, edge_attr, lin_in_W, lin_in_b, g1_Wl, g1_bl, g1_Wr,
           g1_br, g1_We, g1_att, g1_bias, res1_W, res1_b, bn1_g, bn1_b, g2_Wl,
           g2_bl, g2_Wr, g2_br, g2_We, g2_att, g2_bias, res2_W, res2_b, bn2_g,
           bn2_b, out_W, out_b):
    src = edge_index[0]
    dst = edge_index[1]
    f32 = jnp.float32
    attv1 = g1_att.reshape(1, F1)
    attv2 = g2_att.reshape(1, HC)
    s4 = jnp.concatenate(
        [jnp.repeat(jnp.eye(H, dtype=f32), HC, axis=0),
         jnp.zeros((F1, 8 - H), f32)], axis=1)
    s4t = s4.T
    ones8 = jnp.ones((CH_DEG, 8), f32)
    z16 = jnp.zeros((NPW, ED), f32)
    z8 = jnp.zeros((NPW, 8), f32)
    z128 = jnp.zeros((NPW, 128), f32)
    znp8 = jnp.zeros((NP, 8), f32)

    h0 = _k_in(x, lin_in_W, lin_in_b.reshape(1, HC))
    outa, outd = _k_deg(dst, edge_attr, ones8, z16, z8)

    xl1, xr1 = _k_lin(h0, g1_Wl, g1_bl.reshape(1, F1), g1_Wr, g1_br.reshape(1, F1))
    xj1, xi1 = _k_gather(xl1, xr1, src, dst, F1, CH_G1)
    p1, s1 = _k_edge1(xj1, xi1, edge_attr, g1_We, attv1, s4, s4t)
    acc1 = _k_scat1(p1, dst, z128)
    den1 = _k_scatd(s1, dst, znp8)
    h1 = _k_post1(h0, xl1, xr1, acc1[0], acc1[1], den1,
                  outa[0], outa[1], outd[0], outd[1], g1_We, attv1, s4, s4t,
                  g1_bias.reshape(1, F1), res1_W, res1_b.reshape(1, F1),
                  bn1_g.reshape(1, F1), bn1_b.reshape(1, F1))

    comb = _k_lin2(h1, g2_Wl, g2_bl.reshape(1, HC), g2_Wr, g2_br.reshape(1, HC))
    xjx, xix = _k_gather(comb, comb, src, dst, 2 * HC, CH_G2)
    p2 = _k_edge2(xjx, xix, edge_attr, g2_We, attv2)
    acc2 = _k_scat2(p2, dst, z128)
    out = _k_post2(h1, comb, acc2[0], acc2[1], outa[0], outa[1],
                   outd[0], outd[1], g2_We, attv2,
                   g2_bias.reshape(1, HC), res2_W, res2_b.reshape(1, HC),
                   bn2_g.reshape(1, HC), bn2_b.reshape(1, HC),
                   out_W.reshape(1, HC), out_b.reshape(1, 1))
    return out


# speedup vs baseline: 25.5335x; 1.1506x over previous
import functools

import jax
import jax.numpy as jnp
from jax import lax
from jax.experimental import pallas as pl
from jax.experimental.pallas import tpu as pltpu
from jax.experimental.pallas import tpu_sc as plsc

N, E, F_IN, ED, H, HC = 10000, 320000, 128, 16, 4, 64
F1 = H * HC
W1 = 128
WD = 8
W2 = 128
NC, NS = 2, 16
NP = 10240
NPW = NP // NS
EPW = E // (NC * NS)
EPW1 = E // NS
CH_DEG, CH_G1, CH_G2, CH_S1, CH_SD, CH_S2 = 1000, 200, 400, 160, 1000, 80
RSQ = 1.0 / (1.0 + 1e-5) ** 0.5


def _mesh():
    return plsc.VectorSubcoreMesh(
        core_axis_name="c", subcore_axis_name="s",
        num_cores=NC, num_subcores=NS)


_SC_PARAMS = pltpu.CompilerParams(use_tc_tiling_on_sc=False,
                                  needs_layout_passes=False)
_SC_PARAMS_T = pltpu.CompilerParams(use_tc_tiling_on_sc=True)


def _k_in_body(x_ref, w_ref, b_ref, o_ref):
    o_ref[...] = jax.nn.relu(
        jnp.dot(x_ref[...], w_ref[...], preferred_element_type=jnp.float32)
        + b_ref[...])


def _k_in(x, w, b):
    return pl.pallas_call(
        _k_in_body,
        out_shape=jax.ShapeDtypeStruct((N, HC), jnp.float32),
        grid=(25,),
        in_specs=[pl.BlockSpec((400, F_IN), lambda i: (i, 0)),
                  pl.BlockSpec((F_IN, HC), lambda i: (0, 0)),
                  pl.BlockSpec((1, HC), lambda i: (0, 0))],
        out_specs=pl.BlockSpec((400, HC), lambda i: (i, 0)),
    )(x, w, b)


def _k_lin_body(h_ref, wl_ref, bl_ref, wr_ref, br_ref, xl_ref, xr_ref):
    h = h_ref[...]
    xl_ref[...] = jnp.dot(h, wl_ref[...], preferred_element_type=jnp.float32) + bl_ref[...]
    xr_ref[...] = jnp.dot(h, wr_ref[...], preferred_element_type=jnp.float32) + br_ref[...]


def _k_lin(h, wl, bl, wr, br):
    fin, fout = wl.shape
    return pl.pallas_call(
        _k_lin_body,
        out_shape=(jax.ShapeDtypeStruct((N, fout), jnp.float32),
                   jax.ShapeDtypeStruct((N, fout), jnp.float32)),
        grid=(25,),
        in_specs=[pl.BlockSpec((400, fin), lambda i: (i, 0)),
                  pl.BlockSpec((fin, fout), lambda i: (0, 0)),
                  pl.BlockSpec((1, fout), lambda i: (0, 0)),
                  pl.BlockSpec((fin, fout), lambda i: (0, 0)),
                  pl.BlockSpec((1, fout), lambda i: (0, 0))],
        out_specs=(pl.BlockSpec((400, fout), lambda i: (i, 0)),
                   pl.BlockSpec((400, fout), lambda i: (i, 0))),
    )(h, wl, bl, wr, br)


def _k_lin2_body(h_ref, wl_ref, bl_ref, wr_ref, br_ref, o_ref):
    h = h_ref[...]
    xl = jnp.dot(h, wl_ref[...], preferred_element_type=jnp.float32) + bl_ref[...]
    xr = jnp.dot(h, wr_ref[...], preferred_element_type=jnp.float32) + br_ref[...]
    o_ref[...] = jnp.concatenate([xl, xr], axis=1)


def _k_lin2(h, wl, bl, wr, br):
    return pl.pallas_call(
        _k_lin2_body,
        out_shape=jax.ShapeDtypeStruct((N, 2 * HC), jnp.float32),
        grid=(25,),
        in_specs=[pl.BlockSpec((400, F1), lambda i: (i, 0)),
                  pl.BlockSpec((F1, HC), lambda i: (0, 0)),
                  pl.BlockSpec((1, HC), lambda i: (0, 0)),
                  pl.BlockSpec((F1, HC), lambda i: (0, 0)),
                  pl.BlockSpec((1, HC), lambda i: (0, 0))],
        out_specs=pl.BlockSpec((400, 2 * HC), lambda i: (i, 0)),
    )(h, wl, bl, wr, br)


def _k_edge1_body(xj_ref, xi_ref, ea_ref, we_ref, attv_ref, s4_ref, s4t_ref, o_ref, s_ref):
    xj = xj_ref[...]
    ee = jnp.dot(ea_ref[...], we_ref[...], preferred_element_type=jnp.float32)
    t = xj + xi_ref[...] + ee
    t = jnp.where(t >= 0, t, 0.2 * t)
    ta = t * attv_ref[...]
    al = jnp.dot(ta, s4_ref[...], preferred_element_type=jnp.float32)
    s = jnp.exp(al)
    pw = jnp.dot(s, s4t_ref[...], preferred_element_type=jnp.float32) * xj
    o_ref[0] = pw[:, 0:128]
    o_ref[1] = pw[:, 128:256]
    dn = (((0,), (1,)), ((), ()))
    s_ref[0] = jnp.exp(lax.dot_general(
        s4_ref[...], ta[0:1280], dn, preferred_element_type=jnp.float32))
    s_ref[1] = jnp.exp(lax.dot_general(
        s4_ref[...], ta[1280:2560], dn, preferred_element_type=jnp.float32))


def _k_edge1(xj, xi, ea, we, attv, s4, s4t):
    eb = 2560
    return pl.pallas_call(
        _k_edge1_body,
        out_shape=(jax.ShapeDtypeStruct((NC, E, W1), jnp.float32),
                   jax.ShapeDtypeStruct((E // 1280, 8, 1280), jnp.float32)),
        grid=(E // eb,),
        in_specs=[pl.BlockSpec((eb, F1), lambda i: (i, 0)),
                  pl.BlockSpec((eb, F1), lambda i: (i, 0)),
                  pl.BlockSpec((eb, ED), lambda i: (i, 0)),
                  pl.BlockSpec((ED, F1), lambda i: (0, 0)),
                  pl.BlockSpec((1, F1), lambda i: (0, 0)),
                  pl.BlockSpec((F1, 8), lambda i: (0, 0)),
                  pl.BlockSpec((8, F1), lambda i: (0, 0))],
        out_specs=(pl.BlockSpec((NC, eb, W1), lambda i: (0, i, 0)),
                   pl.BlockSpec((2, 8, 1280), lambda i: (i, 0, 0))),
    )(xj, xi, ea, we, attv, s4, s4t)


def _k_edge2_body(xjx_ref, xix_ref, ea_ref, we_ref, attv_ref, o_ref):
    xj = xjx_ref[:, 0:HC]
    xi = xix_ref[:, HC:2 * HC]
    ee = jnp.dot(ea_ref[...], we_ref[...], preferred_element_type=jnp.float32)
    t = xj + xi + ee
    t = jnp.where(t >= 0, t, 0.2 * t)
    al = jnp.sum(t * attv_ref[...], axis=1, keepdims=True)
    s = jnp.exp(al)
    pw = s * xj
    eb = pw.shape[0]
    pad = jnp.zeros((eb, W2 - HC - 1), jnp.float32)
    o_ref[...] = jnp.concatenate([pw, s, pad], axis=1)


def _k_edge2(xj, xi, ea, we, attv):
    eb = 8000
    return pl.pallas_call(
        _k_edge2_body,
        out_shape=jax.ShapeDtypeStruct((E, W2), jnp.float32),
        grid=(E // eb,),
        in_specs=[pl.BlockSpec((eb, 2 * HC), lambda i: (i, 0)),
                  pl.BlockSpec((eb, 2 * HC), lambda i: (i, 0)),
                  pl.BlockSpec((eb, ED), lambda i: (i, 0)),
                  pl.BlockSpec((ED, HC), lambda i: (0, 0)),
                  pl.BlockSpec((1, HC), lambda i: (0, 0))],
        out_specs=pl.BlockSpec((eb, W2), lambda i: (i, 0)),
    )(xj, xi, ea, we, attv)


def _loop_attr_blk(aa0, aa1, ad0, ad1):
    deg = ad0[:, 0:1] + ad1[:, 0:1]
    return (aa0 + aa1) / jnp.maximum(deg, 1.0)


def _k_post1_body(h0_ref, xl_ref, xr_ref, a0_ref, a1_ref, dsum_ref,
                  aa0_ref, aa1_ref, ad0_ref, ad1_ref, we_ref, attv_ref,
                  s4_ref, s4t_ref, bias_ref, rw_ref, rb_ref,
                  bg_ref, bb_ref, o_ref):
    xl = xl_ref[...]
    la = _loop_attr_blk(aa0_ref[...], aa1_ref[...], ad0_ref[...], ad1_ref[...])
    eel = jnp.dot(la, we_ref[...], preferred_element_type=jnp.float32)
    t = xl + xr_ref[...] + eel
    t = jnp.where(t >= 0, t, 0.2 * t)
    al = jnp.dot(t * attv_ref[...], s4_ref[...], preferred_element_type=jnp.float32)
    sl = jnp.exp(al)
    num = jnp.concatenate([a0_ref[...], a1_ref[...]], axis=1)
    num = num + jnp.dot(sl, s4t_ref[...], preferred_element_type=jnp.float32) * xl
    dtot = jnp.sum(dsum_ref[...], axis=0) + sl
    denr = jnp.dot(dtot, s4t_ref[...], preferred_element_type=jnp.float32)
    gat = num / denr + bias_ref[...]
    res = jnp.dot(h0_ref[...], rw_ref[...], preferred_element_type=jnp.float32) + rb_ref[...]
    o_ref[...] = jax.nn.relu(bg_ref[...] * (gat + res) * RSQ + bb_ref[...])


def _k_post1(h0, xl1, xr1, a0, a1, dsum, aa0, aa1, ad0, ad1, we, attv,
             s4, s4t, bias, rw, rb, bg, bb):
    nb = 400
    return pl.pallas_call(
        _k_post1_body,
        out_shape=jax.ShapeDtypeStruct((N, F1), jnp.float32),
        grid=(N // nb,),
        in_specs=[pl.BlockSpec((nb, HC), lambda i: (i, 0)),
                  pl.BlockSpec((nb, F1), lambda i: (i, 0)),
                  pl.BlockSpec((nb, F1), lambda i: (i, 0)),
                  pl.BlockSpec((nb, W1), lambda i: (i, 0)),
                  pl.BlockSpec((nb, W1), lambda i: (i, 0)),
                  pl.BlockSpec((NC * NS, nb, 8), lambda i: (0, i, 0)),
                  pl.BlockSpec((nb, ED), lambda i: (i, 0)),
                  pl.BlockSpec((nb, ED), lambda i: (i, 0)),
                  pl.BlockSpec((nb, 8), lambda i: (i, 0)),
                  pl.BlockSpec((nb, 8), lambda i: (i, 0)),
                  pl.BlockSpec((ED, F1), lambda i: (0, 0)),
                  pl.BlockSpec((1, F1), lambda i: (0, 0)),
                  pl.BlockSpec((F1, 8), lambda i: (0, 0)),
                  pl.BlockSpec((8, F1), lambda i: (0, 0)),
                  pl.BlockSpec((1, F1), lambda i: (0, 0)),
                  pl.BlockSpec((HC, F1), lambda i: (0, 0)),
                  pl.BlockSpec((1, F1), lambda i: (0, 0)),
                  pl.BlockSpec((1, F1), lambda i: (0, 0)),
                  pl.BlockSpec((1, F1), lambda i: (0, 0))],
        out_specs=pl.BlockSpec((nb, F1), lambda i: (i, 0)),
    )(h0, xl1, xr1, a0, a1, dsum, aa0, aa1, ad0, ad1,
      we, attv, s4, s4t, bias, rw, rb, bg, bb)


def _k_post2_body(h1_ref, comb_ref, a0_ref, a1_ref, aa0_ref, aa1_ref,
                  ad0_ref, ad1_ref, we_ref, attv_ref, bias_ref, rw_ref, rb_ref,
                  bg_ref, bb_ref, ow_ref, ob_ref, o_ref):
    xl = comb_ref[:, 0:HC]
    la = _loop_attr_blk(aa0_ref[...], aa1_ref[...], ad0_ref[...], ad1_ref[...])
    eel = jnp.dot(la, we_ref[...], preferred_element_type=jnp.float32)
    t = xl + comb_ref[:, HC:2 * HC] + eel
    t = jnp.where(t >= 0, t, 0.2 * t)
    al = jnp.sum(t * attv_ref[...], axis=1, keepdims=True)
    sl = jnp.exp(al)
    a0 = a0_ref[...]
    a1 = a1_ref[...]
    num = a0[:, 0:HC] + a1[:, 0:HC] + sl * xl
    den = a0[:, HC:HC + 1] + a1[:, HC:HC + 1] + sl
    gat = num / den + bias_ref[...]
    res = jnp.dot(h1_ref[...], rw_ref[...], preferred_element_type=jnp.float32) + rb_ref[...]
    h2 = jax.nn.relu(bg_ref[...] * (gat + res) * RSQ + bb_ref[...])
    o_ref[...] = jnp.sum(h2 * ow_ref[...], axis=1, keepdims=True) + ob_ref[...]


def _k_post2(h1, comb, a0, a1, aa0, aa1, ad0, ad1, we, attv, bias,
             rw, rb, bg, bb, ow, ob):
    nb = 400
    return pl.pallas_call(
        _k_post2_body,
        out_shape=jax.ShapeDtypeStruct((N, 1), jnp.float32),
        grid=(N // nb,),
        in_specs=[pl.BlockSpec((nb, F1), lambda i: (i, 0)),
                  pl.BlockSpec((nb, 2 * HC), lambda i: (i, 0)),
                  pl.BlockSpec((nb, W2), lambda i: (i, 0)),
                  pl.BlockSpec((nb, W2), lambda i: (i, 0)),
                  pl.BlockSpec((nb, ED), lambda i: (i, 0)),
                  pl.BlockSpec((nb, ED), lambda i: (i, 0)),
                  pl.BlockSpec((nb, 8), lambda i: (i, 0)),
                  pl.BlockSpec((nb, 8), lambda i: (i, 0)),
                  pl.BlockSpec((ED, HC), lambda i: (0, 0)),
                  pl.BlockSpec((1, HC), lambda i: (0, 0)),
                  pl.BlockSpec((1, HC), lambda i: (0, 0)),
                  pl.BlockSpec((F1, HC), lambda i: (0, 0)),
                  pl.BlockSpec((1, HC), lambda i: (0, 0)),
                  pl.BlockSpec((1, HC), lambda i: (0, 0)),
                  pl.BlockSpec((1, HC), lambda i: (0, 0)),
                  pl.BlockSpec((1, HC), lambda i: (0, 0)),
                  pl.BlockSpec((1, 1), lambda i: (0, 0))],
        out_specs=pl.BlockSpec((nb, 1), lambda i: (i, 0)),
    )(h1, comb, a0, a1, aa0, aa1, ad0, ad1,
      we, attv, bias, rw, rb, bg, bb, ow, ob)


def _deg_body(dst_hbm, ea_hbm, ones_hbm, z16_hbm, z8_hbm, outa_hbm, outd_hbm,
              dst_v, ea_v, ones_v, acc_a, acc_d):
    c = lax.axis_index("c")
    s = lax.axis_index("s")
    wid = c * NS + s
    pltpu.sync_copy(z16_hbm, acc_a.at[pl.ds(s * NPW, NPW)])
    pltpu.sync_copy(z8_hbm, acc_d.at[pl.ds(s * NPW, NPW)])
    pltpu.sync_copy(ones_hbm, ones_v)
    plsc.subcore_barrier()

    def body(i, carry):
        base = wid * EPW + i * CH_DEG
        pltpu.sync_copy(dst_hbm.at[pl.ds(base, CH_DEG)], dst_v)
        pltpu.sync_copy(ea_hbm.at[pl.ds(base, CH_DEG)], ea_v)
        pltpu.sync_copy(ea_v, acc_a.at[dst_v], add=True)
        pltpu.sync_copy(ones_v, acc_d.at[dst_v], add=True)
        return carry

    lax.fori_loop(0, EPW // CH_DEG, body, 0)
    plsc.subcore_barrier()
    pltpu.sync_copy(acc_a.at[pl.ds(s * NPW, NPW)],
                    outa_hbm.at[c, pl.ds(s * NPW, NPW)])
    pltpu.sync_copy(acc_d.at[pl.ds(s * NPW, NPW)],
                    outd_hbm.at[c, pl.ds(s * NPW, NPW)])


def _k_deg(dst, ea, ones8, z16, z8):
    return pl.kernel(
        _deg_body,
        out_type=(jax.ShapeDtypeStruct((NC, NP, ED), jnp.float32),
                  jax.ShapeDtypeStruct((NC, NP, 8), jnp.float32)),
        mesh=_mesh(),
        compiler_params=_SC_PARAMS,
        scratch_types=[pltpu.VMEM((CH_DEG,), jnp.int32),
                       pltpu.VMEM((CH_DEG, ED), jnp.float32),
                       pltpu.VMEM((CH_DEG, 8), jnp.float32),
                       pltpu.VMEM_SHARED((NP, ED), jnp.float32),
                       pltpu.VMEM_SHARED((NP, 8), jnp.float32)],
    )(dst, ea, ones8, z16, z8)


def _gather_body(ch, xl_hbm, xr_hbm, src_hbm, dst_hbm, xj_hbm, xi_hbm,
                 src_all, dst_all, rows_a, rows_b, sem_a, sem_b):
    c = lax.axis_index("c")
    s = lax.axis_index("s")
    wid = c * NS + s
    e0 = wid * EPW
    pltpu.sync_copy(src_hbm.at[pl.ds(e0, EPW)], src_all)
    pltpu.sync_copy(dst_hbm.at[pl.ds(e0, EPW)], dst_all)
    n = EPW // ch

    def start_j(i):
        pltpu.async_copy(xl_hbm.at[src_all.at[pl.ds(i * ch, ch)]], rows_a, sem_a)

    def start_i(i):
        pltpu.async_copy(xr_hbm.at[dst_all.at[pl.ds(i * ch, ch)]], rows_b, sem_b)

    def wait_j(i):
        pltpu.make_async_copy(
            xl_hbm.at[src_all.at[pl.ds(i * ch, ch)]], rows_a, sem_a).wait()

    def wait_i(i):
        pltpu.make_async_copy(
            xr_hbm.at[dst_all.at[pl.ds(i * ch, ch)]], rows_b, sem_b).wait()

    start_j(0)

    def body(i, carry):
        wait_j(i)
        start_i(i)
        pltpu.sync_copy(rows_a, xj_hbm.at[pl.ds(e0 + i * ch, ch)])
        pl.when(i + 1 < n)(lambda: start_j(i + 1))
        wait_i(i)
        pltpu.sync_copy(rows_b, xi_hbm.at[pl.ds(e0 + i * ch, ch)])
        return carry

    lax.fori_loop(0, n, body, 0)


def _k_gather(xl, xr, src, dst, f, ch):
    return pl.kernel(
        functools.partial(_gather_body, ch),
        out_type=(jax.ShapeDtypeStruct((E, f), jnp.float32),
                  jax.ShapeDtypeStruct((E, f), jnp.float32)),
        mesh=_mesh(),
        compiler_params=_SC_PARAMS_T,
        scratch_types=[pltpu.VMEM((EPW,), jnp.int32),
                       pltpu.VMEM((EPW,), jnp.int32),
                       pltpu.VMEM((ch, f), jnp.float32),
                       pltpu.VMEM((ch, f), jnp.float32),
                       pltpu.SemaphoreType.DMA,
                       pltpu.SemaphoreType.DMA],
    )(xl, xr, src, dst)


def _scat1_body(p_hbm, dst_hbm, z_hbm, out_hbm,
                dst_a, dst_b, p_a, p_b, sem_a, sem_b, acc):
    c = lax.axis_index("c")
    s = lax.axis_index("s")
    pltpu.sync_copy(z_hbm, acc.at[pl.ds(s * NPW, NPW)])
    plsc.subcore_barrier()
    n = EPW1 // CH_S1

    def start(dv, pv, sem, chunk):
        base = s * EPW1 + chunk * CH_S1
        pltpu.async_copy(dst_hbm.at[pl.ds(base, CH_S1)], dv, sem)
        pltpu.async_copy(p_hbm.at[c, pl.ds(base, CH_S1)], pv, sem)

    def wait(dv, pv, sem, chunk):
        base = s * EPW1 + chunk * CH_S1
        pltpu.make_async_copy(dst_hbm.at[pl.ds(base, CH_S1)], dv, sem).wait()
        pltpu.make_async_copy(p_hbm.at[c, pl.ds(base, CH_S1)], pv, sem).wait()

    start(dst_a, p_a, sem_a, 0)

    def body(i, carry):
        c0 = 2 * i
        start(dst_b, p_b, sem_b, c0 + 1)
        wait(dst_a, p_a, sem_a, c0)
        pltpu.sync_copy(p_a, acc.at[dst_a], add=True)
        pl.when(c0 + 2 < n)(lambda: start(dst_a, p_a, sem_a, c0 + 2))
        wait(dst_b, p_b, sem_b, c0 + 1)
        pltpu.sync_copy(p_b, acc.at[dst_b], add=True)
        return carry

    lax.fori_loop(0, n // 2, body, 0)
    wait(dst_a, p_a, sem_a, n - 1)
    pltpu.sync_copy(p_a, acc.at[dst_a], add=True)
    plsc.subcore_barrier()
    pltpu.sync_copy(acc.at[pl.ds(s * NPW, NPW)],
                    out_hbm.at[c, pl.ds(s * NPW, NPW)])


def _k_scat1(p, dst, z128):
    return pl.kernel(
        _scat1_body,
        out_type=jax.ShapeDtypeStruct((NC, NP, W1), jnp.float32),
        mesh=_mesh(),
        compiler_params=_SC_PARAMS_T,
        scratch_types=[pltpu.VMEM((CH_S1,), jnp.int32),
                       pltpu.VMEM((CH_S1,), jnp.int32),
                       pltpu.VMEM((CH_S1, W1), jnp.float32),
                       pltpu.VMEM((CH_S1, W1), jnp.float32),
                       pltpu.SemaphoreType.DMA,
                       pltpu.SemaphoreType.DMA,
                       pltpu.VMEM_SHARED((NP, W1), jnp.float32)],
    )(p, dst, z128)


def _scatd_body(spk_hbm, dst_hbm, z_hbm, out_hbm, dst_v, s_stage, acc2d):
    c = lax.axis_index("c")
    s = lax.axis_index("s")
    wid = c * NS + s
    pltpu.sync_copy(z_hbm, acc2d)
    nblk = jnp.where(wid < 26, 8, 7)
    b0 = jnp.where(wid < 26, 8 * wid, 208 + 7 * (wid - 26))
    lanes = lax.broadcasted_iota(jnp.int32, (16,), 0)

    def blk(i, carry):
        b = b0 + i
        pltpu.sync_copy(spk_hbm.at[b], s_stage)
        pltpu.sync_copy(dst_hbm.at[pl.ds(b * 1280, 1280)], dst_v)

        def row(r, carry2):
            dstv = dst_v[pl.ds(r * 16, 16)]
            for h in range(H):
                sv = plsc.load_gather(
                    s_stage, [jnp.full((16,), h, jnp.int32), r * 16 + lanes])
                plsc.addupdate_scatter(
                    acc2d, [dstv, jnp.full((16,), h, jnp.int32)], sv)
            return carry2

        lax.fori_loop(0, 80, row, 0)
        return carry

    lax.fori_loop(0, nblk, blk, 0)
    pltpu.sync_copy(acc2d, out_hbm.at[wid])


def _k_scatd(spk, dst, z):
    return pl.kernel(
        _scatd_body,
        out_type=jax.ShapeDtypeStruct((NC * NS, NP, 8), jnp.float32),
        mesh=_mesh(),
        compiler_params=_SC_PARAMS,
        scratch_types=[pltpu.VMEM((1280,), jnp.int32),
                       pltpu.VMEM((8, 1280), jnp.float32),
                       pltpu.VMEM((NP, 8), jnp.float32)],
    )(spk, dst, z)


def _scat2_body(p_hbm, dst_hbm, z_hbm, out_hbm,
                dst_a, dst_b, p_a, p_b, sem_a, sem_b, acc):
    c = lax.axis_index("c")
    s = lax.axis_index("s")
    wid = c * NS + s
    pltpu.sync_copy(z_hbm, acc.at[pl.ds(s * NPW, NPW)])
    plsc.subcore_barrier()
    n = EPW // CH_S2

    def start(dv, pv, sem, chunk):
        base = wid * EPW + chunk * CH_S2
        pltpu.async_copy(dst_hbm.at[pl.ds(base, CH_S2)], dv, sem)
        pltpu.async_copy(p_hbm.at[pl.ds(base, CH_S2)], pv, sem)

    def wait(dv, pv, sem, chunk):
        base = wid * EPW + chunk * CH_S2
        pltpu.make_async_copy(dst_hbm.at[pl.ds(base, CH_S2)], dv, sem).wait()
        pltpu.make_async_copy(p_hbm.at[pl.ds(base, CH_S2)], pv, sem).wait()

    start(dst_a, p_a, sem_a, 0)

    def body(i, carry):
        c0 = 2 * i
        start(dst_b, p_b, sem_b, c0 + 1)
        wait(dst_a, p_a, sem_a, c0)
        pltpu.sync_copy(p_a, acc.at[dst_a], add=True)
        pl.when(c0 + 2 < n)(lambda: start(dst_a, p_a, sem_a, c0 + 2))
        wait(dst_b, p_b, sem_b, c0 + 1)
        pltpu.sync_copy(p_b, acc.at[dst_b], add=True)
        return carry

    lax.fori_loop(0, n // 2, body, 0)
    wait(dst_a, p_a, sem_a, n - 1)
    pltpu.sync_copy(p_a, acc.at[dst_a], add=True)
    plsc.subcore_barrier()
    pltpu.sync_copy(acc.at[pl.ds(s * NPW, NPW)],
                    out_hbm.at[c, pl.ds(s * NPW, NPW)])


def _k_scat2(p, dst, z128):
    return pl.kernel(
        _scat2_body,
        out_type=jax.ShapeDtypeStruct((NC, NP, W2), jnp.float32),
        mesh=_mesh(),
        compiler_params=_SC_PARAMS_T,
        scratch_types=[pltpu.VMEM((CH_S2,), jnp.int32),
                       pltpu.VMEM((CH_S2,), jnp.int32),
                       pltpu.VMEM((CH_S2, W2), jnp.float32),
                       pltpu.VMEM((CH_S2, W2), jnp.float32),
                       pltpu.SemaphoreType.DMA,
                       pltpu.SemaphoreType.DMA,
                       pltpu.VMEM_SHARED((NP, W2), jnp.float32)],
    )(p, dst, z128)


def kernel(x, edge_index, edge_attr, lin_in_W, lin_in_b, g1_Wl, g1_bl, g1_Wr,
           g1_br, g1_We, g1_att, g1_bias, res1_W, res1_b, bn1_g, bn1_b, g2_Wl,
           g2_bl, g2_Wr, g2_br, g2_We, g2_att, g2_bias, res2_W, res2_b, bn2_g,
           bn2_b, out_W, out_b):
    src = edge_index[0]
    dst = edge_index[1]
    f32 = jnp.float32
    attv1 = g1_att.reshape(1, F1)
    attv2 = g2_att.reshape(1, HC)
    s4 = jnp.concatenate(
        [jnp.repeat(jnp.eye(H, dtype=f32), HC, axis=0),
         jnp.zeros((F1, 8 - H), f32)], axis=1)
    s4t = s4.T
    ones8 = jnp.ones((CH_DEG, 8), f32)
    z16 = jnp.zeros((NPW, ED), f32)
    z8 = jnp.zeros((NPW, 8), f32)
    z128 = jnp.zeros((NPW, 128), f32)
    znp8 = jnp.zeros((NP, 8), f32)

    h0 = _k_in(x, lin_in_W, lin_in_b.reshape(1, HC))
    outa, outd = _k_deg(dst, edge_attr, ones8, z16, z8)

    xl1, xr1 = _k_lin(h0, g1_Wl, g1_bl.reshape(1, F1), g1_Wr, g1_br.reshape(1, F1))
    xj1, xi1 = _k_gather(xl1, xr1, src, dst, F1, CH_G1)
    p1, s1 = _k_edge1(xj1, xi1, edge_attr, g1_We, attv1, s4, s4t)
    acc1 = _k_scat1(p1, dst, z128)
    den1 = _k_scatd(s1, dst, znp8)
    h1 = _k_post1(h0, xl1, xr1, acc1[0], acc1[1], den1,
                  outa[0], outa[1], outd[0], outd[1], g1_We, attv1, s4, s4t,
                  g1_bias.reshape(1, F1), res1_W, res1_b.reshape(1, F1),
                  bn1_g.reshape(1, F1), bn1_b.reshape(1, F1))

    comb = _k_lin2(h1, g2_Wl, g2_bl.reshape(1, HC), g2_Wr, g2_br.reshape(1, HC))
    xjx, xix = _k_gather(comb, comb, src, dst, 2 * HC, CH_G2)
    p2 = _k_edge2(xjx, xix, edge_attr, g2_We, attv2)
    acc2 = _k_scat2(p2, dst, z128)
    out = _k_post2(h1, comb, acc2[0], acc2[1], outa[0], outa[1],
                   outd[0], outd[1], g2_We, attv2,
                   g2_bias.reshape(1, HC), res2_W, res2_b.reshape(1, HC),
                   bn2_g.reshape(1, HC), bn2_b.reshape(1, HC),
                   out_W.reshape(1, HC), out_b.reshape(1, 1))
    return out


# speedup vs baseline: 29.4220x; 1.1523x over previous
import functools

import jax
import jax.numpy as jnp
from jax import lax
from jax.experimental import pallas as pl
from jax.experimental.pallas import tpu as pltpu
from jax.experimental.pallas import tpu_sc as plsc

N, E, F_IN, ED, H, HC = 10000, 320000, 128, 16, 4, 64
F1 = H * HC
W1 = 128
WD = 8
W2 = 128
NC, NS = 2, 16
NP = 10240
NPW = NP // NS
EPW = E // (NC * NS)
EPW1 = E // NS
CH_DEG, CH_G1, CH_G2, CH_S1, CH_SD, CH_S2 = 1000, 200, 400, 160, 1000, 80
RSQ = 1.0 / (1.0 + 1e-5) ** 0.5


def _mesh():
    return plsc.VectorSubcoreMesh(
        core_axis_name="c", subcore_axis_name="s",
        num_cores=NC, num_subcores=NS)


_SC_PARAMS = pltpu.CompilerParams(use_tc_tiling_on_sc=False,
                                  needs_layout_passes=False)
_SC_PARAMS_T = pltpu.CompilerParams(use_tc_tiling_on_sc=True)


def _pack2(x):
    lo = jax.lax.bitcast_convert_type(
        x[:, 0:128].astype(jnp.bfloat16), jnp.uint16).astype(jnp.uint32)
    hi = jax.lax.bitcast_convert_type(
        x[:, 128:256].astype(jnp.bfloat16), jnp.uint16).astype(jnp.uint32)
    return ((hi << 16) | lo).astype(jnp.int32)


def _unpack2(w):
    wu = w.astype(jnp.uint32)
    lo = jax.lax.bitcast_convert_type(
        (wu & jnp.uint32(0xFFFF)).astype(jnp.uint16), jnp.bfloat16)
    hi = jax.lax.bitcast_convert_type(
        (wu >> 16).astype(jnp.uint16), jnp.bfloat16)
    return lo.astype(jnp.float32), hi.astype(jnp.float32)


def _k_in_body(x_ref, w_ref, b_ref, o_ref):
    o_ref[...] = jax.nn.relu(
        jnp.dot(x_ref[...], w_ref[...], preferred_element_type=jnp.float32)
        + b_ref[...])


def _k_in(x, w, b):
    return pl.pallas_call(
        _k_in_body,
        out_shape=jax.ShapeDtypeStruct((N, HC), jnp.float32),
        grid=(25,),
        in_specs=[pl.BlockSpec((400, F_IN), lambda i: (i, 0)),
                  pl.BlockSpec((F_IN, HC), lambda i: (0, 0)),
                  pl.BlockSpec((1, HC), lambda i: (0, 0))],
        out_specs=pl.BlockSpec((400, HC), lambda i: (i, 0)),
    )(x, w, b)


def _k_lin_body(h_ref, wl_ref, bl_ref, wr_ref, br_ref, xl_ref, xr_ref):
    h = h_ref[...]
    xl = jnp.dot(h, wl_ref[...], preferred_element_type=jnp.float32) + bl_ref[...]
    xr = jnp.dot(h, wr_ref[...], preferred_element_type=jnp.float32) + br_ref[...]
    xl_ref[...] = _pack2(xl)
    xr_ref[...] = _pack2(xr)


def _k_lin(h, wl, bl, wr, br):
    fin, fout = wl.shape
    return pl.pallas_call(
        _k_lin_body,
        out_shape=(jax.ShapeDtypeStruct((N, 128), jnp.int32),
                   jax.ShapeDtypeStruct((N, 128), jnp.int32)),
        grid=(25,),
        in_specs=[pl.BlockSpec((400, fin), lambda i: (i, 0)),
                  pl.BlockSpec((fin, fout), lambda i: (0, 0)),
                  pl.BlockSpec((1, fout), lambda i: (0, 0)),
                  pl.BlockSpec((fin, fout), lambda i: (0, 0)),
                  pl.BlockSpec((1, fout), lambda i: (0, 0))],
        out_specs=(pl.BlockSpec((400, 128), lambda i: (i, 0)),
                   pl.BlockSpec((400, 128), lambda i: (i, 0))),
    )(h, wl, bl, wr, br)


def _k_lin2_body(h_ref, wl_ref, bl_ref, wr_ref, br_ref, o_ref):
    h = h_ref[...]
    xl = jnp.dot(h, wl_ref[...], preferred_element_type=jnp.float32) + bl_ref[...]
    xr = jnp.dot(h, wr_ref[...], preferred_element_type=jnp.float32) + br_ref[...]
    o_ref[...] = jnp.concatenate([xl, xr], axis=1)


def _k_lin2(h, wl, bl, wr, br):
    return pl.pallas_call(
        _k_lin2_body,
        out_shape=jax.ShapeDtypeStruct((N, 2 * HC), jnp.float32),
        grid=(25,),
        in_specs=[pl.BlockSpec((400, F1), lambda i: (i, 0)),
                  pl.BlockSpec((F1, HC), lambda i: (0, 0)),
                  pl.BlockSpec((1, HC), lambda i: (0, 0)),
                  pl.BlockSpec((F1, HC), lambda i: (0, 0)),
                  pl.BlockSpec((1, HC), lambda i: (0, 0))],
        out_specs=pl.BlockSpec((400, 2 * HC), lambda i: (i, 0)),
    )(h, wl, bl, wr, br)


def _k_edge1_body(xj_ref, xi_ref, ea_ref, we_ref, attv_ref, s4_ref, s4t_ref, o_ref, s_ref):
    xj0, xj1 = _unpack2(xj_ref[...])
    xi0, xi1 = _unpack2(xi_ref[...])
    ee = jnp.dot(ea_ref[...], we_ref[...], preferred_element_type=jnp.float32)
    attv = attv_ref[...]
    t0 = xj0 + xi0 + ee[:, 0:128]
    t0 = jnp.where(t0 >= 0, t0, 0.2 * t0) * attv[:, 0:128]
    t1 = xj1 + xi1 + ee[:, 128:256]
    t1 = jnp.where(t1 >= 0, t1, 0.2 * t1) * attv[:, 128:256]
    ta = jnp.concatenate([t0, t1], axis=1)
    al = jnp.dot(ta, s4_ref[...], preferred_element_type=jnp.float32)
    s = jnp.exp(al)
    srep = jnp.dot(s, s4t_ref[...], preferred_element_type=jnp.float32)
    o_ref[0] = srep[:, 0:128] * xj0
    o_ref[1] = srep[:, 128:256] * xj1
    dn = (((0,), (1,)), ((), ()))
    s_ref[0] = jnp.exp(lax.dot_general(
        s4_ref[...], ta[0:1280], dn, preferred_element_type=jnp.float32))
    s_ref[1] = jnp.exp(lax.dot_general(
        s4_ref[...], ta[1280:2560], dn, preferred_element_type=jnp.float32))


def _k_edge1(xj, xi, ea, we, attv, s4, s4t):
    eb = 2560
    return pl.pallas_call(
        _k_edge1_body,
        out_shape=(jax.ShapeDtypeStruct((NC, E, W1), jnp.float32),
                   jax.ShapeDtypeStruct((E // 1280, 8, 1280), jnp.float32)),
        grid=(E // eb,),
        in_specs=[pl.BlockSpec((eb, 128), lambda i: (i, 0)),
                  pl.BlockSpec((eb, 128), lambda i: (i, 0)),
                  pl.BlockSpec((eb, ED), lambda i: (i, 0)),
                  pl.BlockSpec((ED, F1), lambda i: (0, 0)),
                  pl.BlockSpec((1, F1), lambda i: (0, 0)),
                  pl.BlockSpec((F1, 8), lambda i: (0, 0)),
                  pl.BlockSpec((8, F1), lambda i: (0, 0))],
        out_specs=(pl.BlockSpec((NC, eb, W1), lambda i: (0, i, 0)),
                   pl.BlockSpec((2, 8, 1280), lambda i: (i, 0, 0))),
    )(xj, xi, ea, we, attv, s4, s4t)


def _k_edge2_body(xjx_ref, xix_ref, ea_ref, we_ref, attv_ref, o_ref):
    xj = xjx_ref[:, 0:HC]
    xi = xix_ref[:, HC:2 * HC]
    ee = jnp.dot(ea_ref[...], we_ref[...], preferred_element_type=jnp.float32)
    t = xj + xi + ee
    t = jnp.where(t >= 0, t, 0.2 * t)
    al = jnp.sum(t * attv_ref[...], axis=1, keepdims=True)
    s = jnp.exp(al)
    pw = s * xj
    eb = pw.shape[0]
    pad = jnp.zeros((eb, W2 - HC - 1), jnp.float32)
    o_ref[...] = jnp.concatenate([pw, s, pad], axis=1)


def _k_edge2(xj, xi, ea, we, attv):
    eb = 8000
    return pl.pallas_call(
        _k_edge2_body,
        out_shape=jax.ShapeDtypeStruct((E, W2), jnp.float32),
        grid=(E // eb,),
        in_specs=[pl.BlockSpec((eb, 2 * HC), lambda i: (i, 0)),
                  pl.BlockSpec((eb, 2 * HC), lambda i: (i, 0)),
                  pl.BlockSpec((eb, ED), lambda i: (i, 0)),
                  pl.BlockSpec((ED, HC), lambda i: (0, 0)),
                  pl.BlockSpec((1, HC), lambda i: (0, 0))],
        out_specs=pl.BlockSpec((eb, W2), lambda i: (i, 0)),
    )(xj, xi, ea, we, attv)


def _loop_attr_blk(aa0, aa1, ad0, ad1):
    deg = ad0[:, 0:1] + ad1[:, 0:1]
    return (aa0 + aa1) / jnp.maximum(deg, 1.0)


def _k_post1_body(h0_ref, xl_ref, xr_ref, a0_ref, a1_ref, dsum_ref,
                  aa0_ref, aa1_ref, ad0_ref, ad1_ref, we_ref, attv_ref,
                  s4_ref, s4t_ref, bias_ref, rw_ref, rb_ref,
                  bg_ref, bb_ref, o_ref):
    xl0, xl1 = _unpack2(xl_ref[...])
    xr0, xr1 = _unpack2(xr_ref[...])
    xl = jnp.concatenate([xl0, xl1], axis=1)
    xr = jnp.concatenate([xr0, xr1], axis=1)
    la = _loop_attr_blk(aa0_ref[...], aa1_ref[...], ad0_ref[...], ad1_ref[...])
    eel = jnp.dot(la, we_ref[...], preferred_element_type=jnp.float32)
    t = xl + xr + eel
    t = jnp.where(t >= 0, t, 0.2 * t)
    al = jnp.dot(t * attv_ref[...], s4_ref[...], preferred_element_type=jnp.float32)
    sl = jnp.exp(al)
    num = jnp.concatenate([a0_ref[...], a1_ref[...]], axis=1)
    num = num + jnp.dot(sl, s4t_ref[...], preferred_element_type=jnp.float32) * xl
    dtot = jnp.sum(dsum_ref[...], axis=0) + sl
    denr = jnp.dot(dtot, s4t_ref[...], preferred_element_type=jnp.float32)
    gat = num / denr + bias_ref[...]
    res = jnp.dot(h0_ref[...], rw_ref[...], preferred_element_type=jnp.float32) + rb_ref[...]
    o_ref[...] = jax.nn.relu(bg_ref[...] * (gat + res) * RSQ + bb_ref[...])


def _k_post1(h0, xl1, xr1, a0, a1, dsum, aa0, aa1, ad0, ad1, we, attv,
             s4, s4t, bias, rw, rb, bg, bb):
    nb = 400
    return pl.pallas_call(
        _k_post1_body,
        out_shape=jax.ShapeDtypeStruct((N, F1), jnp.float32),
        grid=(N // nb,),
        in_specs=[pl.BlockSpec((nb, HC), lambda i: (i, 0)),
                  pl.BlockSpec((nb, 128), lambda i: (i, 0)),
                  pl.BlockSpec((nb, 128), lambda i: (i, 0)),
                  pl.BlockSpec((nb, W1), lambda i: (i, 0)),
                  pl.BlockSpec((nb, W1), lambda i: (i, 0)),
                  pl.BlockSpec((NC * NS, nb, 8), lambda i: (0, i, 0)),
                  pl.BlockSpec((nb, ED), lambda i: (i, 0)),
                  pl.BlockSpec((nb, ED), lambda i: (i, 0)),
                  pl.BlockSpec((nb, 8), lambda i: (i, 0)),
                  pl.BlockSpec((nb, 8), lambda i: (i, 0)),
                  pl.BlockSpec((ED, F1), lambda i: (0, 0)),
                  pl.BlockSpec((1, F1), lambda i: (0, 0)),
                  pl.BlockSpec((F1, 8), lambda i: (0, 0)),
                  pl.BlockSpec((8, F1), lambda i: (0, 0)),
                  pl.BlockSpec((1, F1), lambda i: (0, 0)),
                  pl.BlockSpec((HC, F1), lambda i: (0, 0)),
                  pl.BlockSpec((1, F1), lambda i: (0, 0)),
                  pl.BlockSpec((1, F1), lambda i: (0, 0)),
                  pl.BlockSpec((1, F1), lambda i: (0, 0))],
        out_specs=pl.BlockSpec((nb, F1), lambda i: (i, 0)),
    )(h0, xl1, xr1, a0, a1, dsum, aa0, aa1, ad0, ad1,
      we, attv, s4, s4t, bias, rw, rb, bg, bb)


def _k_post2_body(h1_ref, comb_ref, a0_ref, a1_ref, aa0_ref, aa1_ref,
                  ad0_ref, ad1_ref, we_ref, attv_ref, bias_ref, rw_ref, rb_ref,
                  bg_ref, bb_ref, ow_ref, ob_ref, o_ref):
    xl = comb_ref[:, 0:HC]
    la = _loop_attr_blk(aa0_ref[...], aa1_ref[...], ad0_ref[...], ad1_ref[...])
    eel = jnp.dot(la, we_ref[...], preferred_element_type=jnp.float32)
    t = xl + comb_ref[:, HC:2 * HC] + eel
    t = jnp.where(t >= 0, t, 0.2 * t)
    al = jnp.sum(t * attv_ref[...], axis=1, keepdims=True)
    sl = jnp.exp(al)
    a0 = a0_ref[...]
    a1 = a1_ref[...]
    num = a0[:, 0:HC] + a1[:, 0:HC] + sl * xl
    den = a0[:, HC:HC + 1] + a1[:, HC:HC + 1] + sl
    gat = num / den + bias_ref[...]
    res = jnp.dot(h1_ref[...], rw_ref[...], preferred_element_type=jnp.float32) + rb_ref[...]
    h2 = jax.nn.relu(bg_ref[...] * (gat + res) * RSQ + bb_ref[...])
    o_ref[...] = jnp.sum(h2 * ow_ref[...], axis=1, keepdims=True) + ob_ref[...]


def _k_post2(h1, comb, a0, a1, aa0, aa1, ad0, ad1, we, attv, bias,
             rw, rb, bg, bb, ow, ob):
    nb = 400
    return pl.pallas_call(
        _k_post2_body,
        out_shape=jax.ShapeDtypeStruct((N, 1), jnp.float32),
        grid=(N // nb,),
        in_specs=[pl.BlockSpec((nb, F1), lambda i: (i, 0)),
                  pl.BlockSpec((nb, 2 * HC), lambda i: (i, 0)),
                  pl.BlockSpec((nb, W2), lambda i: (i, 0)),
                  pl.BlockSpec((nb, W2), lambda i: (i, 0)),
                  pl.BlockSpec((nb, ED), lambda i: (i, 0)),
                  pl.BlockSpec((nb, ED), lambda i: (i, 0)),
                  pl.BlockSpec((nb, 8), lambda i: (i, 0)),
                  pl.BlockSpec((nb, 8), lambda i: (i, 0)),
                  pl.BlockSpec((ED, HC), lambda i: (0, 0)),
                  pl.BlockSpec((1, HC), lambda i: (0, 0)),
                  pl.BlockSpec((1, HC), lambda i: (0, 0)),
                  pl.BlockSpec((F1, HC), lambda i: (0, 0)),
                  pl.BlockSpec((1, HC), lambda i: (0, 0)),
                  pl.BlockSpec((1, HC), lambda i: (0, 0)),
                  pl.BlockSpec((1, HC), lambda i: (0, 0)),
                  pl.BlockSpec((1, HC), lambda i: (0, 0)),
                  pl.BlockSpec((1, 1), lambda i: (0, 0))],
        out_specs=pl.BlockSpec((nb, 1), lambda i: (i, 0)),
    )(h1, comb, a0, a1, aa0, aa1, ad0, ad1,
      we, attv, bias, rw, rb, bg, bb, ow, ob)


def _deg_body(dst_hbm, ea_hbm, ones_hbm, z16_hbm, z8_hbm, outa_hbm, outd_hbm,
              dst_v, ea_v, ones_v, acc_a, acc_d):
    c = lax.axis_index("c")
    s = lax.axis_index("s")
    wid = c * NS + s
    pltpu.sync_copy(z16_hbm, acc_a.at[pl.ds(s * NPW, NPW)])
    pltpu.sync_copy(z8_hbm, acc_d.at[pl.ds(s * NPW, NPW)])
    pltpu.sync_copy(ones_hbm, ones_v)
    plsc.subcore_barrier()

    def body(i, carry):
        base = wid * EPW + i * CH_DEG
        pltpu.sync_copy(dst_hbm.at[pl.ds(base, CH_DEG)], dst_v)
        pltpu.sync_copy(ea_hbm.at[pl.ds(base, CH_DEG)], ea_v)
        pltpu.sync_copy(ea_v, acc_a.at[dst_v], add=True)
        pltpu.sync_copy(ones_v, acc_d.at[dst_v], add=True)
        return carry

    lax.fori_loop(0, EPW // CH_DEG, body, 0)
    plsc.subcore_barrier()
    pltpu.sync_copy(acc_a.at[pl.ds(s * NPW, NPW)],
                    outa_hbm.at[c, pl.ds(s * NPW, NPW)])
    pltpu.sync_copy(acc_d.at[pl.ds(s * NPW, NPW)],
                    outd_hbm.at[c, pl.ds(s * NPW, NPW)])


def _k_deg(dst, ea, ones8, z16, z8):
    return pl.kernel(
        _deg_body,
        out_type=(jax.ShapeDtypeStruct((NC, NP, ED), jnp.float32),
                  jax.ShapeDtypeStruct((NC, NP, 8), jnp.float32)),
        mesh=_mesh(),
        compiler_params=_SC_PARAMS,
        scratch_types=[pltpu.VMEM((CH_DEG,), jnp.int32),
                       pltpu.VMEM((CH_DEG, ED), jnp.float32),
                       pltpu.VMEM((CH_DEG, 8), jnp.float32),
                       pltpu.VMEM_SHARED((NP, ED), jnp.float32),
                       pltpu.VMEM_SHARED((NP, 8), jnp.float32)],
    )(dst, ea, ones8, z16, z8)


def _gather_body(ch, xl_hbm, xr_hbm, src_hbm, dst_hbm, xj_hbm, xi_hbm,
                 src_all, dst_all, rows_a, rows_b, sem_a, sem_b):
    c = lax.axis_index("c")
    s = lax.axis_index("s")
    wid = c * NS + s
    e0 = wid * EPW
    pltpu.sync_copy(src_hbm.at[pl.ds(e0, EPW)], src_all)
    pltpu.sync_copy(dst_hbm.at[pl.ds(e0, EPW)], dst_all)
    n = EPW // ch

    def start_j(i):
        pltpu.async_copy(xl_hbm.at[src_all.at[pl.ds(i * ch, ch)]], rows_a, sem_a)

    def start_i(i):
        pltpu.async_copy(xr_hbm.at[dst_all.at[pl.ds(i * ch, ch)]], rows_b, sem_b)

    def wait_j(i):
        pltpu.make_async_copy(
            xl_hbm.at[src_all.at[pl.ds(i * ch, ch)]], rows_a, sem_a).wait()

    def wait_i(i):
        pltpu.make_async_copy(
            xr_hbm.at[dst_all.at[pl.ds(i * ch, ch)]], rows_b, sem_b).wait()

    start_j(0)

    def body(i, carry):
        wait_j(i)
        start_i(i)
        pltpu.sync_copy(rows_a, xj_hbm.at[pl.ds(e0 + i * ch, ch)])
        pl.when(i + 1 < n)(lambda: start_j(i + 1))
        wait_i(i)
        pltpu.sync_copy(rows_b, xi_hbm.at[pl.ds(e0 + i * ch, ch)])
        return carry

    lax.fori_loop(0, n, body, 0)


def _k_gather(xl, xr, src, dst, row_shape, dtype, ch):
    return pl.kernel(
        functools.partial(_gather_body, ch),
        out_type=(jax.ShapeDtypeStruct((E,) + row_shape, dtype),
                  jax.ShapeDtypeStruct((E,) + row_shape, dtype)),
        mesh=_mesh(),
        compiler_params=_SC_PARAMS_T,
        scratch_types=[pltpu.VMEM((EPW,), jnp.int32),
                       pltpu.VMEM((EPW,), jnp.int32),
                       pltpu.VMEM((ch,) + row_shape, dtype),
                       pltpu.VMEM((ch,) + row_shape, dtype),
                       pltpu.SemaphoreType.DMA,
                       pltpu.SemaphoreType.DMA],
    )(xl, xr, src, dst)


def _scat1_body(p_hbm, dst_hbm, z_hbm, out_hbm,
                dst_a, dst_b, p_a, p_b, sem_a, sem_b, acc):
    c = lax.axis_index("c")
    s = lax.axis_index("s")
    pltpu.sync_copy(z_hbm, acc.at[pl.ds(s * NPW, NPW)])
    plsc.subcore_barrier()
    n = EPW1 // CH_S1

    def start(dv, pv, sem, chunk):
        base = s * EPW1 + chunk * CH_S1
        pltpu.async_copy(dst_hbm.at[pl.ds(base, CH_S1)], dv, sem)
        pltpu.async_copy(p_hbm.at[c, pl.ds(base, CH_S1)], pv, sem)

    def wait(dv, pv, sem, chunk):
        base = s * EPW1 + chunk * CH_S1
        pltpu.make_async_copy(dst_hbm.at[pl.ds(base, CH_S1)], dv, sem).wait()
        pltpu.make_async_copy(p_hbm.at[c, pl.ds(base, CH_S1)], pv, sem).wait()

    start(dst_a, p_a, sem_a, 0)

    def body(i, carry):
        c0 = 2 * i
        start(dst_b, p_b, sem_b, c0 + 1)
        wait(dst_a, p_a, sem_a, c0)
        pltpu.sync_copy(p_a, acc.at[dst_a], add=True)
        pl.when(c0 + 2 < n)(lambda: start(dst_a, p_a, sem_a, c0 + 2))
        wait(dst_b, p_b, sem_b, c0 + 1)
        pltpu.sync_copy(p_b, acc.at[dst_b], add=True)
        return carry

    lax.fori_loop(0, n // 2, body, 0)
    wait(dst_a, p_a, sem_a, n - 1)
    pltpu.sync_copy(p_a, acc.at[dst_a], add=True)
    plsc.subcore_barrier()
    pltpu.sync_copy(acc.at[pl.ds(s * NPW, NPW)],
                    out_hbm.at[c, pl.ds(s * NPW, NPW)])


def _k_scat1(p, dst, z128):
    return pl.kernel(
        _scat1_body,
        out_type=jax.ShapeDtypeStruct((NC, NP, W1), jnp.float32),
        mesh=_mesh(),
        compiler_params=_SC_PARAMS_T,
        scratch_types=[pltpu.VMEM((CH_S1,), jnp.int32),
                       pltpu.VMEM((CH_S1,), jnp.int32),
                       pltpu.VMEM((CH_S1, W1), jnp.float32),
                       pltpu.VMEM((CH_S1, W1), jnp.float32),
                       pltpu.SemaphoreType.DMA,
                       pltpu.SemaphoreType.DMA,
                       pltpu.VMEM_SHARED((NP, W1), jnp.float32)],
    )(p, dst, z128)


def _scatd_body(spk_hbm, dst_hbm, z_hbm, out_hbm, dst_v, s_stage, acc2d):
    c = lax.axis_index("c")
    s = lax.axis_index("s")
    wid = c * NS + s
    pltpu.sync_copy(z_hbm, acc2d)
    nblk = jnp.where(wid < 26, 8, 7)
    b0 = jnp.where(wid < 26, 8 * wid, 208 + 7 * (wid - 26))
    lanes = lax.broadcasted_iota(jnp.int32, (16,), 0)

    def blk(i, carry):
        b = b0 + i
        pltpu.sync_copy(spk_hbm.at[b], s_stage)
        pltpu.sync_copy(dst_hbm.at[pl.ds(b * 1280, 1280)], dst_v)

        def row(r, carry2):
            dstv = dst_v[pl.ds(r * 16, 16)]
            for h in range(H):
                sv = plsc.load_gather(
                    s_stage, [jnp.full((16,), h, jnp.int32), r * 16 + lanes])
                plsc.addupdate_scatter(
                    acc2d, [dstv, jnp.full((16,), h, jnp.int32)], sv)
            return carry2

        lax.fori_loop(0, 80, row, 0)
        return carry

    lax.fori_loop(0, nblk, blk, 0)
    pltpu.sync_copy(acc2d, out_hbm.at[wid])


def _k_scatd(spk, dst, z):
    return pl.kernel(
        _scatd_body,
        out_type=jax.ShapeDtypeStruct((NC * NS, NP, 8), jnp.float32),
        mesh=_mesh(),
        compiler_params=_SC_PARAMS,
        scratch_types=[pltpu.VMEM((1280,), jnp.int32),
                       pltpu.VMEM((8, 1280), jnp.float32),
                       pltpu.VMEM((NP, 8), jnp.float32)],
    )(spk, dst, z)


def _scat2_body(p_hbm, dst_hbm, z_hbm, out_hbm,
                dst_a, dst_b, p_a, p_b, sem_a, sem_b, acc):
    c = lax.axis_index("c")
    s = lax.axis_index("s")
    wid = c * NS + s
    pltpu.sync_copy(z_hbm, acc.at[pl.ds(s * NPW, NPW)])
    plsc.subcore_barrier()
    n = EPW // CH_S2

    def start(dv, pv, sem, chunk):
        base = wid * EPW + chunk * CH_S2
        pltpu.async_copy(dst_hbm.at[pl.ds(base, CH_S2)], dv, sem)
        pltpu.async_copy(p_hbm.at[pl.ds(base, CH_S2)], pv, sem)

    def wait(dv, pv, sem, chunk):
        base = wid * EPW + chunk * CH_S2
        pltpu.make_async_copy(dst_hbm.at[pl.ds(base, CH_S2)], dv, sem).wait()
        pltpu.make_async_copy(p_hbm.at[pl.ds(base, CH_S2)], pv, sem).wait()

    start(dst_a, p_a, sem_a, 0)

    def body(i, carry):
        c0 = 2 * i
        start(dst_b, p_b, sem_b, c0 + 1)
        wait(dst_a, p_a, sem_a, c0)
        pltpu.sync_copy(p_a, acc.at[dst_a], add=True)
        pl.when(c0 + 2 < n)(lambda: start(dst_a, p_a, sem_a, c0 + 2))
        wait(dst_b, p_b, sem_b, c0 + 1)
        pltpu.sync_copy(p_b, acc.at[dst_b], add=True)
        return carry

    lax.fori_loop(0, n // 2, body, 0)
    wait(dst_a, p_a, sem_a, n - 1)
    pltpu.sync_copy(p_a, acc.at[dst_a], add=True)
    plsc.subcore_barrier()
    pltpu.sync_copy(acc.at[pl.ds(s * NPW, NPW)],
                    out_hbm.at[c, pl.ds(s * NPW, NPW)])


def _k_scat2(p, dst, z128):
    return pl.kernel(
        _scat2_body,
        out_type=jax.ShapeDtypeStruct((NC, NP, W2), jnp.float32),
        mesh=_mesh(),
        compiler_params=_SC_PARAMS_T,
        scratch_types=[pltpu.VMEM((CH_S2,), jnp.int32),
                       pltpu.VMEM((CH_S2,), jnp.int32),
                       pltpu.VMEM((CH_S2, W2), jnp.float32),
                       pltpu.VMEM((CH_S2, W2), jnp.float32),
                       pltpu.SemaphoreType.DMA,
                       pltpu.SemaphoreType.DMA,
                       pltpu.VMEM_SHARED((NP, W2), jnp.float32)],
    )(p, dst, z128)


def kernel(x, edge_index, edge_attr, lin_in_W, lin_in_b, g1_Wl, g1_bl, g1_Wr,
           g1_br, g1_We, g1_att, g1_bias, res1_W, res1_b, bn1_g, bn1_b, g2_Wl,
           g2_bl, g2_Wr, g2_br, g2_We, g2_att, g2_bias, res2_W, res2_b, bn2_g,
           bn2_b, out_W, out_b):
    src = edge_index[0]
    dst = edge_index[1]
    f32 = jnp.float32
    attv1 = g1_att.reshape(1, F1)
    attv2 = g2_att.reshape(1, HC)
    s4 = jnp.concatenate(
        [jnp.repeat(jnp.eye(H, dtype=f32), HC, axis=0),
         jnp.zeros((F1, 8 - H), f32)], axis=1)
    s4t = s4.T
    ones8 = jnp.ones((CH_DEG, 8), f32)
    z16 = jnp.zeros((NPW, ED), f32)
    z8 = jnp.zeros((NPW, 8), f32)
    z128 = jnp.zeros((NPW, 128), f32)
    znp8 = jnp.zeros((NP, 8), f32)

    h0 = _k_in(x, lin_in_W, lin_in_b.reshape(1, HC))
    outa, outd = _k_deg(dst, edge_attr, ones8, z16, z8)

    xl1, xr1 = _k_lin(h0, g1_Wl, g1_bl.reshape(1, F1), g1_Wr, g1_br.reshape(1, F1))
    xj1, xi1 = _k_gather(xl1, xr1, src, dst, (128,), jnp.int32, CH_G1)
    p1, s1 = _k_edge1(xj1, xi1, edge_attr, g1_We, attv1, s4, s4t)
    acc1 = _k_scat1(p1, dst, z128)
    den1 = _k_scatd(s1, dst, znp8)
    h1 = _k_post1(h0, xl1, xr1, acc1[0], acc1[1], den1,
                  outa[0], outa[1], outd[0], outd[1], g1_We, attv1, s4, s4t,
                  g1_bias.reshape(1, F1), res1_W, res1_b.reshape(1, F1),
                  bn1_g.reshape(1, F1), bn1_b.reshape(1, F1))

    comb = _k_lin2(h1, g2_Wl, g2_bl.reshape(1, HC), g2_Wr, g2_br.reshape(1, HC))
    xjx, xix = _k_gather(comb, comb, src, dst, (2 * HC,), jnp.float32, CH_G2)
    p2 = _k_edge2(xjx, xix, edge_attr, g2_We, attv2)
    acc2 = _k_scat2(p2, dst, z128)
    out = _k_post2(h1, comb, acc2[0], acc2[1], outa[0], outa[1],
                   outd[0], outd[1], g2_We, attv2,
                   g2_bias.reshape(1, HC), res2_W, res2_b.reshape(1, HC),
                   bn2_g.reshape(1, HC), bn2_b.reshape(1, HC),
                   out_W.reshape(1, HC), out_b.reshape(1, 1))
    return out
